# Initial kernel scaffold; baseline (speedup 1.0000x reference)
import jax
import jax.numpy as jnp
from jax.experimental import pallas as pl


def kernel(x, edge_index, batch, Wl1, Wr1, att1, b1, Wl2, Wr2, att2, b2, W3, b3):
    raise NotImplementedError("write your pallas kernel here")


# speedup vs baseline: 1.0056x; 1.0056x over previous
import functools

import jax
import jax.numpy as jnp
from jax.experimental import pallas as pl
from jax.experimental.pallas import tpu as pltpu

_N = 10000
_NP = 10240
_HEADS = 10
_OUT = 64
_HID = 640
_NG = 256


def _mm_body(x_ref, w_ref, o_ref):
    o_ref[...] = jnp.dot(x_ref[...], w_ref[...],
                         preferred_element_type=jnp.float32)


def _matmul(x, w, bm=512):
    m, k = x.shape
    _, n = w.shape
    grid = (m // bm,)
    return pl.pallas_call(
        _mm_body,
        grid=grid,
        in_specs=[
            pl.BlockSpec((bm, k), lambda i: (i, 0)),
            pl.BlockSpec((k, n), lambda i: (0, 0)),
        ],
        out_specs=pl.BlockSpec((bm, n), lambda i: (i, 0)),
        out_shape=jax.ShapeDtypeStruct((m, n), jnp.float32),
    )(x, w)


def _gatv2(x, src, dst, Wl, Wr, att, bias, heads, out_ch):
    n = x.shape[0]
    xl = _matmul(x, Wl).reshape(n, heads, out_ch)
    xr = _matmul(x, Wr).reshape(n, heads, out_ch)
    e = jax.nn.leaky_relu(xl[src] + xr[dst], 0.2)
    logits = jnp.sum(e * att[None, :, :], axis=-1)
    m = jax.ops.segment_max(logits, dst, num_segments=n)
    m = jnp.where(jnp.isfinite(m), m, 0.0)
    p = jnp.exp(logits - m[dst])
    denom = jax.ops.segment_sum(p, dst, num_segments=n)
    alpha = p / (denom[dst] + 1e-16)
    out = jax.ops.segment_sum(xl[src] * alpha[:, :, None], dst, num_segments=n)
    return out.reshape(n, heads * out_ch) + bias


def kernel(x, edge_index, batch, Wl1, Wr1, att1, b1, Wl2, Wr2, att2, b2, W3, b3):
    n = x.shape[0]
    xpad = jnp.pad(x, ((0, _NP - n), (0, 0)))
    sl = jnp.arange(n, dtype=edge_index.dtype)
    src = jnp.concatenate([edge_index[0], sl])
    dst = jnp.concatenate([edge_index[1], sl])

    h = jax.nn.elu(_gatv2(xpad, src, dst, Wl1, Wr1, att1, b1, _HEADS, _OUT)[:n])
    hpad = jnp.pad(h, ((0, _NP - n), (0, 0)))
    h = _gatv2(hpad, src, dst, Wl2, Wr2, att2, b2, 1, _HID)[:n]

    hpad = jnp.pad(h, ((0, _NP - n), (0, 0)))
    deg = jax.ops.segment_sum(jnp.ones(src.shape, x.dtype), dst, num_segments=n)
    dinv = 1.0 / jnp.sqrt(jnp.clip(deg, 1.0))
    hw = _matmul(hpad, W3)[:n]
    norm = dinv[src] * dinv[dst]
    h = jax.ops.segment_sum(hw[src] * norm[:, None], dst, num_segments=n) + b3
    h = jax.nn.relu(h)

    cnt = jax.ops.segment_sum(jnp.ones((n,), h.dtype), batch, num_segments=_NG)
    gmax = jax.ops.segment_max(h, batch, num_segments=_NG)
    gmax = jnp.where(cnt[:, None] > 0, gmax, 0.0)
    gmean = jax.ops.segment_sum(h, batch, num_segments=_NG) / jnp.clip(cnt, 1.0)[:, None]
    return jnp.concatenate([gmax, gmean], axis=1)


# speedup vs baseline: 4.2256x; 4.2022x over previous
import functools

import jax
import jax.numpy as jnp
from jax import lax
from jax.experimental import pallas as pl
from jax.experimental.pallas import tpu as pltpu
from jax.experimental.pallas import tpu_sc as plsc

N = 10000
NP = 10240
E0 = 320000
EP = 331776
HID = 640
NF = 5
NG = 256
BM = 512
def _mesh():
    return plsc.VectorSubcoreMesh(core_axis_name="c", subcore_axis_name="s")
_OWNER = (0, 0, 1, 1, 0)


def _mm_chunked(xc, wr, bias=None, act=None, out_scale=None):
    kc = xc.shape[0]
    grid = (NF, NP // BM, kc)

    def body(*refs):
        if bias is not None and out_scale is not None:
            x_ref, w_ref, b_ref, s_ref, o_ref = refs
        elif bias is not None:
            x_ref, w_ref, b_ref, o_ref = refs
            s_ref = None
        elif out_scale is not None:
            x_ref, w_ref, s_ref, o_ref = refs
            b_ref = None
        else:
            x_ref, w_ref, o_ref = refs
            b_ref = s_ref = None
        i = pl.program_id(1)
        ki = pl.program_id(2)
        x = x_ref[0]
        if b_ref is not None:
            x = x + b_ref[0]
        if act == "elu":
            x = jnp.where(x > 0, x, jnp.exp(x) - 1.0)
        acc = jnp.dot(x, w_ref[0, 0],
                      preferred_element_type=jnp.float32)

        @pl.when(ki == 0)
        def _():
            o_ref[0] = jnp.zeros_like(o_ref[0])

        o_ref[0] += acc

        @pl.when(ki == kc - 1)
        def _():
            r = o_ref[0]
            if s_ref is not None:
                r = r * s_ref[...]
            rows = i * BM + lax.broadcasted_iota(jnp.int32, (BM, 128), 0)
            o_ref[0] = jnp.where(rows < N, r, 0.0)

    specs = [
        pl.BlockSpec((1, BM, 128), lambda j, i, ki: (ki, i, 0)),
        pl.BlockSpec((1, 1, 128, 128), lambda j, i, ki: (ki, j, 0, 0)),
    ]
    args = [xc, wr]
    if bias is not None:
        specs.append(pl.BlockSpec((1, 1, 128), lambda j, i, ki: (ki, 0, 0)))
        args.append(bias.reshape(-1, 1, 128))
    if out_scale is not None:
        specs.append(pl.BlockSpec((BM, 128), lambda j, i, ki: (i, 0)))
        args.append(out_scale)
    return pl.pallas_call(
        body, grid=grid, in_specs=specs,
        out_specs=pl.BlockSpec((1, BM, 128), lambda j, i, ki: (j, i, 0)),
        out_shape=jax.ShapeDtypeStruct((NF, NP, 128), jnp.float32),
    )(*args)


def _bounds(xc, attc, heads):
    def body(x_ref, a_ref, o_ref):
        cols = []
        if heads == 1:
            tot = jnp.zeros((BM,), jnp.float32)
            for f in range(NF):
                t = x_ref[f] * a_ref[f][None, :]
                tot = tot + jnp.sum(jnp.maximum(t, 0.2 * t), axis=-1)
            cols.append(tot)
        else:
            for h in range(heads):
                f, sl = (64 * h) // 128, (64 * h) % 128
                t = x_ref[f, :, sl:sl + 64] * a_ref[f, sl:sl + 64][None, :]
                cols.append(jnp.sum(jnp.maximum(t, 0.2 * t), axis=-1))
        stacked = jnp.stack(cols, axis=-1)
        o_ref[...] = jnp.concatenate(
            [stacked, jnp.zeros((BM, 128 - len(cols)), jnp.float32)],
            axis=-1)

    return pl.pallas_call(
        body, grid=(NP // BM,),
        in_specs=[pl.BlockSpec((NF, BM, 128), lambda i: (0, i, 0)),
                  pl.BlockSpec((NF, 128), lambda i: (0, 0))],
        out_specs=pl.BlockSpec((BM, 128), lambda i: (i, 0)),
        out_shape=jax.ShapeDtypeStruct((NP, 128), jnp.float32),
    )(xc, attc)


def _colmax(u):
    def body(u_ref, o_ref):
        i = pl.program_id(0)

        @pl.when(i == 0)
        def _():
            o_ref[...] = jnp.full((1, 128), -3e38, jnp.float32)

        o_ref[...] = jnp.maximum(o_ref[...],
                                 jnp.max(u_ref[...], axis=0, keepdims=True))

    return pl.pallas_call(
        body, grid=(NP // BM,),
        in_specs=[pl.BlockSpec((BM, 128), lambda i: (i, 0))],
        out_specs=pl.BlockSpec((1, 128), lambda i: (0, 0)),
        out_shape=jax.ShapeDtypeStruct((1, 128), jnp.float32),
    )(u)


def _s0_table(u2, mx):
    def body(u_ref, m_ref, o_ref):
        o_ref[...] = u_ref[...] + m_ref[...]

    return pl.pallas_call(
        body, grid=(NP // BM,),
        in_specs=[pl.BlockSpec((BM, 128), lambda i: (i, 0)),
                  pl.BlockSpec((1, 128), lambda i: (0, 0))],
        out_specs=pl.BlockSpec((BM, 128), lambda i: (i, 0)),
        out_shape=jax.ShapeDtypeStruct((NP, 128), jnp.float32),
    )(u2, mx)


def _lsr_table(d0p, heads, with_dinv):
    n_out = 2 if with_dinv else 1

    def body(d_ref, o_ref, *maybe_dinv):
        d0 = d_ref[0] + d_ref[1]
        ls = jnp.log(jnp.maximum(d0, 1e-35))
        den = d0 / jnp.maximum(d0, 1e-35)
        r = 1.0 / (den + 1e-16)
        colv = lax.broadcasted_iota(jnp.int32, (BM, 128), 1)
        r = jnp.where(colv < heads, r, 0.0)
        o_ref[...] = jnp.concatenate(
            [ls[:, :16], r[:, :16], jnp.zeros((BM, 96), jnp.float32)],
            axis=1)
        if with_dinv:
            dv = maybe_dinv[0]
            deg = d0[:, 10]
            dinv = 1.0 / jnp.sqrt(jnp.maximum(deg, 1.0))
            dv[...] = jnp.broadcast_to(dinv[:, None], (BM, 128))

    out_shape = [jax.ShapeDtypeStruct((NP, 128), jnp.float32)] * n_out
    res = pl.pallas_call(
        body, grid=(NP // BM,),
        in_specs=[pl.BlockSpec((2, BM, 128), lambda i: (0, i, 0))],
        out_specs=[pl.BlockSpec((BM, 128), lambda i: (i, 0))] * n_out,
        out_shape=out_shape,
    )(d0p)
    return res if with_dinv else (res[0],)


def _final_feat(aggc, dinv, b3c):
    def body(a_ref, d_ref, b_ref, o_ref):
        i = pl.program_id(1)
        r = jnp.maximum(a_ref[0] * d_ref[...] + b_ref[0], 0.0)
        rows = i * BM + lax.broadcasted_iota(jnp.int32, (BM, 128), 0)
        o_ref[0] = jnp.where(rows < N, r, 0.0)

    return pl.pallas_call(
        body, grid=(NF, NP // BM),
        in_specs=[pl.BlockSpec((1, BM, 128), lambda f, i: (f, i, 0)),
                  pl.BlockSpec((BM, 128), lambda f, i: (i, 0)),
                  pl.BlockSpec((1, 1, 128), lambda f, i: (f, 0, 0))],
        out_specs=pl.BlockSpec((1, BM, 128), lambda f, i: (f, i, 0)),
        out_shape=jax.ShapeDtypeStruct((NF, NP, 128), jnp.float32),
    )(aggc, dinv, b3c.reshape(NF, 1, 128))


def _graph_tables(batch2d):
    def body(b_ref, c_ref, s_ref):
        i = pl.program_id(0)

        @pl.when(i == 0)
        def _():
            c_ref[...] = jnp.zeros((1, NG), jnp.int32)
            s_ref[...] = jnp.zeros((1, NG), jnp.int32)

        b = b_ref[...]
        g = lax.broadcasted_iota(jnp.int32, (BM, NG), 1)
        c_ref[...] += jnp.sum((b == g).astype(jnp.int32), axis=0,
                              keepdims=True)
        s_ref[...] += jnp.sum((b < g).astype(jnp.int32), axis=0,
                              keepdims=True)

    return pl.pallas_call(
        body, grid=(NP // BM,),
        in_specs=[pl.BlockSpec((BM, 1), lambda i: (i, 0))],
        out_specs=[pl.BlockSpec((1, NG), lambda i: (0, 0))] * 2,
        out_shape=[jax.ShapeDtypeStruct((1, NG), jnp.int32)] * 2,
    )(batch2d)


def _sc_wid():
    return lax.axis_index("s") * 2 + lax.axis_index("c")


def _edge_logits(heads, with_deg):
    CA = 16
    PT = EP // 32
    nch = PT // CA
    ch_head = HID // heads

    scratch = [pltpu.VMEM((HID,), jnp.float32)]
    scratch += [pltpu.VMEM((CA, 128), jnp.float32) for _ in range(10)]
    scratch += [pltpu.VMEM((CA, 128), jnp.float32)]
    scratch += [pltpu.VMEM((CA,), jnp.int32),
                pltpu.VMEM((CA,), jnp.int32),
                pltpu.VMEM((CA, 16), jnp.float32),
                pltpu.VMEM((CA, 128), jnp.float32),
                pltpu.VMEM((16, 128), jnp.float32),
                pltpu.VMEM_SHARED((NP, 128), jnp.float32),
                pltpu.SemaphoreType.DMA]

    @functools.partial(
        pl.kernel,
        out_type=[jax.ShapeDtypeStruct((EP, 16), jnp.float32),
                  jax.ShapeDtypeStruct((2, NP, 128), jnp.float32)],
        mesh=_mesh(),
        compiler_params=pltpu.CompilerParams(needs_layout_passes=False),
        scratch_types=scratch)
    def k(xlc, xrc, s0t, att_h, srca, dsta, l_out, d0_out,
          attv, xb0, xb1, xb2, xb3, xb4, rb0, rb1, rb2, rb3, rb4, s0b,
          idxs, idxd, lbuf, pbuf, zbuf, d0s, sem):
        cc = lax.axis_index("c")
        wid = _sc_wid()
        xlb = [xb0, xb1, xb2, xb3, xb4]
        xrb = [rb0, rb1, rb2, rb3, rb4]
        lane = lax.iota(jnp.int32, 16)

        pltpu.sync_copy(att_h, attv)
        for i in range(16):
            for q in range(8):
                zbuf[i, pl.ds(16 * q, 16)] = jnp.zeros((16,), jnp.float32)
        for b in range(40):
            r0 = lax.axis_index("s") * 640 + b * 16
            pltpu.sync_copy(zbuf, d0s.at[pl.ds(r0, 16), :])
        for i in range(CA):
            for q in range(1, 8):
                pbuf[i, pl.ds(16 * q, 16)] = jnp.zeros((16,), jnp.float32)
        plsc.subcore_barrier()

        tile_base = wid * PT

        def chunk(ci, _):
            base = tile_base + ci * CA
            pltpu.sync_copy(srca.at[pl.ds(base, CA)], idxs)
            pltpu.sync_copy(dsta.at[pl.ds(base, CA)], idxd)
            descs = []
            for f in range(NF):
                descs.append(pltpu.async_copy(xlc.at[f].at[idxs],
                                              xlb[f], sem))
                descs.append(pltpu.async_copy(xrc.at[f].at[idxd],
                                              xrb[f], sem))
            descs.append(pltpu.async_copy(s0t.at[idxd], s0b, sem))
            for d in descs:
                d.wait()

            def edge(i, _):
                tots = []
                for h in range(heads):
                    acc = None
                    for q in range(ch_head // 16):
                        g = h * ch_head + q * 16
                        f, off = g // 128, g % 128
                        a = xlb[f][i, pl.ds(off, 16)]
                        b = xrb[f][i, pl.ds(off, 16)]
                        z = a + b
                        w = jnp.maximum(z, 0.2 * z) * attv[pl.ds(g, 16)]
                        acc = w if acc is None else acc + w
                    tots.append(jnp.sum(acc))
                lrow = jnp.zeros((16,), jnp.float32)
                for h in range(heads):
                    lrow = jnp.where(lane == h,
                                     jnp.full((16,), tots[h], jnp.float32),
                                     lrow)
                s0r = s0b[i, pl.ds(0, 16)]
                hm = lane < heads
                ls = jnp.where(hm, lrow - s0r, 0.0)
                p = jnp.where(hm, jnp.exp(ls), 0.0)
                if with_deg:
                    p = p + jnp.where(lane == 10, 1.0, 0.0)
                lbuf[i, :] = ls
                pbuf[i, pl.ds(0, 16)] = p
                return 0

            lax.fori_loop(0, CA, edge, 0)
            pltpu.sync_copy(lbuf, l_out.at[pl.ds(base, CA), :])
            pltpu.sync_copy(pbuf, d0s.at[idxd], add=True)
            return 0

        lax.fori_loop(0, nch, chunk, 0)
        plsc.subcore_barrier()
        r0 = lax.axis_index("s") * 640
        pltpu.sync_copy(d0s.at[pl.ds(r0, 640), :],
                        d0_out.at[cc].at[pl.ds(r0, 640), :])

    return k


def _alpha_pass():
    CB = 64
    PT = EP // 32
    nch = PT // CB

    @functools.partial(
        pl.kernel,
        out_type=jax.ShapeDtypeStruct((EP, 16), jnp.float32),
        mesh=_mesh(),
        compiler_params=pltpu.CompilerParams(needs_layout_passes=False),
        scratch_types=[pltpu.VMEM((CB, 16), jnp.float32),
                       pltpu.VMEM((CB, 128), jnp.float32),
                       pltpu.VMEM((CB,), jnp.int32),
                       pltpu.VMEM((CB, 16), jnp.float32),
                       pltpu.SemaphoreType.DMA])
    def k(l_in, lsr, dsta, a_out, lb, lsrb, idxd, ab, sem):
        wid = _sc_wid()
        tile_base = wid * PT

        def chunk(ci, _):
            base = tile_base + ci * CB
            pltpu.sync_copy(dsta.at[pl.ds(base, CB)], idxd)
            d1 = pltpu.async_copy(l_in.at[pl.ds(base, CB), :], lb, sem)
            d2 = pltpu.async_copy(lsr.at[idxd], lsrb, sem)
            d1.wait()
            d2.wait()

            def edge(i, _):
                ls = lsrb[i, pl.ds(0, 16)]
                r = lsrb[i, pl.ds(16, 16)]
                ab[i, :] = jnp.exp(lb[i, :] - ls) * r
                return 0

            lax.fori_loop(0, CB, edge, 0)
            pltpu.sync_copy(ab, a_out.at[pl.ds(base, CB), :])
            return 0

        lax.fori_loop(0, nch, chunk, 0)

    return k


def _aggregate(heads):
    CC = 64
    PT = EP // 16
    nch = PT // CC
    has_alpha = heads > 0

    scratch = [pltpu.VMEM((CC, 128), jnp.float32),
               pltpu.VMEM((CC, 128), jnp.float32),
               pltpu.VMEM((CC,), jnp.int32),
               pltpu.VMEM((CC,), jnp.int32),
               pltpu.VMEM((CC, 16), jnp.float32),
               pltpu.VMEM((16, 128), jnp.float32),
               pltpu.VMEM_SHARED((NP, 128), jnp.float32),
               pltpu.SemaphoreType.DMA]

    @functools.partial(
        pl.kernel,
        out_type=jax.ShapeDtypeStruct((NF, NP, 128), jnp.float32),
        mesh=_mesh(),
        compiler_params=pltpu.CompilerParams(needs_layout_passes=False),
        scratch_types=scratch)
    def k(xlc, alpha, srca, dsta, agg,
          xb, vb, idxs, idxd, ab, zbuf, accs, sem):
        cc = lax.axis_index("c")
        sid = lax.axis_index("s")
        tile_base = sid * PT

        for i in range(16):
            for q in range(8):
                zbuf[i, pl.ds(16 * q, 16)] = jnp.zeros((16,), jnp.float32)

        for f in range(NF):
            @pl.when(cc == _OWNER[f])
            def _(f=f):
                for b in range(40):
                    r0 = sid * 640 + b * 16
                    pltpu.sync_copy(zbuf, accs.at[pl.ds(r0, 16), :])
                plsc.subcore_barrier()

                def chunk(ci, _):
                    base = tile_base + ci * CC
                    pltpu.sync_copy(srca.at[pl.ds(base, CC)], idxs)
                    pltpu.sync_copy(dsta.at[pl.ds(base, CC)], idxd)
                    descs = [pltpu.async_copy(xlc.at[f].at[idxs], xb, sem)]
                    if has_alpha:
                        descs.append(pltpu.async_copy(
                            alpha.at[pl.ds(base, CC), :], ab, sem))
                    for d in descs:
                        d.wait()
                    if has_alpha:
                        def edge(i, _):
                            arow = ab[i, :]
                            for q in range(8):
                                h = (f * 128 + q * 16) // (HID // heads)
                                av = jnp.full((16,), arow[h], jnp.float32)
                                vb[i, pl.ds(16 * q, 16)] = (
                                    xb[i, pl.ds(16 * q, 16)] * av)
                            return 0

                        lax.fori_loop(0, CC, edge, 0)
                        pltpu.sync_copy(vb, accs.at[idxd], add=True)
                    else:
                        pltpu.sync_copy(xb, accs.at[idxd], add=True)
                    return 0

                lax.fori_loop(0, nch, chunk, 0)
                plsc.subcore_barrier()
                r0 = sid * 640
                pltpu.sync_copy(accs.at[pl.ds(r0, 640), :],
                                agg.at[f].at[pl.ds(r0, 640), :])
                plsc.subcore_barrier()

    return k


def _pooling():
    @functools.partial(
        pl.kernel,
        out_type=jax.ShapeDtypeStruct((NG, 2 * HID), jnp.float32),
        mesh=_mesh(),
        compiler_params=pltpu.CompilerParams(needs_layout_passes=False),
        scratch_types=[pltpu.VMEM((272,), jnp.int32),
                       pltpu.VMEM((272,), jnp.int32),
                       pltpu.VMEM((8, 128), jnp.float32),
                       pltpu.VMEM((2 * HID,), jnp.float32),
                       pltpu.SemaphoreType.DMA])
    def k(hf, starts_h, cnts_h, out, sv, cv, buf, rowbuf, sem):
        wid = _sc_wid()
        pltpu.sync_copy(starts_h, sv)
        pltpu.sync_copy(cnts_h, cv)
        srow = sv[pl.ds(8 * wid, 16)]
        crow = cv[pl.ds(8 * wid, 16)]
        for j in range(8):
            st = srow[j]
            cn = crow[j]
            st8 = pl.multiple_of(st & (-8), 8)
            dlt = st - st8
            nb = (cn + dlt + 7) >> 3
            cnf = jnp.maximum(cn.astype(jnp.float32), 1.0)
            invv = 1.0 / jnp.full((16,), cnf, jnp.float32)
            nz = cn > 0
            for f in range(NF):
                def blk(b, carry):
                    pltpu.async_copy(
                        hf.at[f].at[pl.ds(st8 + 8 * b, 8), :], buf,
                        sem).wait()
                    out_c = []
                    for q in range(8):
                        mx, sm = carry[2 * q], carry[2 * q + 1]
                        for r in range(8):
                            idx = 8 * b + r
                            valid = (idx >= dlt) & (idx < dlt + cn)
                            vm = jnp.full((16,), valid, jnp.bool_)
                            v = buf[r, pl.ds(16 * q, 16)]
                            mx = jnp.maximum(mx, jnp.where(vm, v, -3e38))
                            sm = sm + jnp.where(vm, v, 0.0)
                        out_c += [mx, sm]
                    return tuple(out_c)

                init = []
                for q in range(8):
                    init += [jnp.full((16,), -3e38, jnp.float32),
                             jnp.zeros((16,), jnp.float32)]
                res = lax.fori_loop(0, nb, blk, tuple(init))
                nzv = jnp.full((16,), nz, jnp.bool_)
                for q in range(8):
                    gmax = jnp.where(nzv, res[2 * q], 0.0)
                    gmax = jnp.where(nzv, gmax, 0.0)
                    gmean = res[2 * q + 1] * invv
                    rowbuf[pl.ds(128 * f + 16 * q, 16)] = gmax
                    rowbuf[pl.ds(HID + 128 * f + 16 * q, 16)] = gmean
            pltpu.sync_copy(rowbuf, out.at[8 * wid + j])

    return k


def _gatv2_layer(xc, src, dst, wl, wr, attf, bias, heads, with_deg,
                 in_bias=None, in_act=None):
    kc = xc.shape[0]
    wl_r = wl.reshape(kc, 128, NF, 128).transpose(0, 2, 1, 3)
    wr_r = wr.reshape(kc, 128, NF, 128).transpose(0, 2, 1, 3)
    xlc = _mm_chunked(xc, wl_r, bias=in_bias, act=in_act)
    xrc = _mm_chunked(xc, wr_r, bias=in_bias, act=in_act)
    attc = attf.reshape(NF, 128)
    u1 = _bounds(xlc, attc, heads)
    u2 = _bounds(xrc, attc, heads)
    mx = _colmax(u1)
    s0 = _s0_table(u2, mx)
    l_e, d0p = _edge_logits(heads, with_deg)(
        xlc, xrc, s0, attf, src, dst)
    tabs = _lsr_table(d0p, heads, with_deg)
    lsr = tabs[0]
    dinv = tabs[1] if with_deg else None
    alpha = _alpha_pass()(l_e, lsr, dst)
    agg = _aggregate(heads)(xlc, alpha, src, dst)
    return agg, dinv


def kernel(x, edge_index, batch, Wl1, Wr1, att1, b1, Wl2, Wr2, att2, b2,
           W3, b3):
    f32 = jnp.float32
    x = x.astype(f32)
    xp = jnp.pad(x, ((0, NP - N), (0, 0))).reshape(1, NP, 128)
    sl = jnp.arange(N, dtype=jnp.int32)
    src = jnp.concatenate([edge_index[0].astype(jnp.int32), sl,
                           jnp.full((EP - E0 - N,), N, jnp.int32)])
    dst = jnp.concatenate([edge_index[1].astype(jnp.int32), sl,
                           jnp.full((EP - E0 - N,), N, jnp.int32)])

    agg1, dinv = _gatv2_layer(xp, src, dst, Wl1, Wr1, att1.reshape(-1),
                              b1, 10, True)
    b1c = b1.reshape(NF, 128)
    agg2, _ = _gatv2_layer(agg1, src, dst, Wl2, Wr2, att2.reshape(-1),
                           b2, 1, False, in_bias=b1c, in_act="elu")
    b2c = b2.reshape(NF, 128)
    w3_r = W3.reshape(NF, 128, NF, 128).transpose(0, 2, 1, 3)
    hws = _mm_chunked(agg2, w3_r, bias=b2c, out_scale=dinv)
    aggg = _aggregate(0)(hws, hws[0, :, :16], src, dst)
    hf = _final_feat(aggg, dinv, b3.reshape(NF, 128))

    batchp = jnp.pad(batch.astype(jnp.int32), (0, NP - N),
                     constant_values=NG)
    cnt, starts = _graph_tables(batchp.reshape(NP, 1))
    starts_p = jnp.pad(starts.reshape(NG), (0, 16))
    cnts_p = jnp.pad(cnt.reshape(NG), (0, 16))
    return _pooling()(hf, starts_p, cnts_p)


# speedup vs baseline: 6.6715x; 1.5788x over previous
import functools

import jax
import jax.numpy as jnp
from jax import lax
from jax.experimental import pallas as pl
from jax.experimental.pallas import tpu as pltpu
from jax.experimental.pallas import tpu_sc as plsc

N = 10000
NP = 10240
E0 = 320000
EP = 331776
HID = 640
NF = 5
NG = 256
BM = 512
def _mesh():
    return plsc.VectorSubcoreMesh(core_axis_name="c", subcore_axis_name="s")
_OWNER = (0, 0, 1, 1, 0)


def _mm_chunked(xc, wr, bias=None, act=None, out_scale=None):
    kc = xc.shape[0]
    grid = (NF, NP // BM, kc)

    def body(*refs):
        if bias is not None and out_scale is not None:
            x_ref, w_ref, b_ref, s_ref, o_ref = refs
        elif bias is not None:
            x_ref, w_ref, b_ref, o_ref = refs
            s_ref = None
        elif out_scale is not None:
            x_ref, w_ref, s_ref, o_ref = refs
            b_ref = None
        else:
            x_ref, w_ref, o_ref = refs
            b_ref = s_ref = None
        i = pl.program_id(1)
        ki = pl.program_id(2)
        x = x_ref[0]
        if b_ref is not None:
            x = x + b_ref[0]
        if act == "elu":
            x = jnp.where(x > 0, x, jnp.exp(x) - 1.0)
        acc = jnp.dot(x, w_ref[0, 0],
                      preferred_element_type=jnp.float32)

        @pl.when(ki == 0)
        def _():
            o_ref[0] = jnp.zeros_like(o_ref[0])

        o_ref[0] += acc

        @pl.when(ki == kc - 1)
        def _():
            r = o_ref[0]
            if s_ref is not None:
                r = r * s_ref[...]
            rows = i * BM + lax.broadcasted_iota(jnp.int32, (BM, 128), 0)
            o_ref[0] = jnp.where(rows < N, r, 0.0)

    specs = [
        pl.BlockSpec((1, BM, 128), lambda j, i, ki: (ki, i, 0)),
        pl.BlockSpec((1, 1, 128, 128), lambda j, i, ki: (ki, j, 0, 0)),
    ]
    args = [xc, wr]
    if bias is not None:
        specs.append(pl.BlockSpec((1, 1, 128), lambda j, i, ki: (ki, 0, 0)))
        args.append(bias.reshape(-1, 1, 128))
    if out_scale is not None:
        specs.append(pl.BlockSpec((BM, 128), lambda j, i, ki: (i, 0)))
        args.append(out_scale)
    return pl.pallas_call(
        body, grid=grid, in_specs=specs,
        out_specs=pl.BlockSpec((1, BM, 128), lambda j, i, ki: (j, i, 0)),
        out_shape=jax.ShapeDtypeStruct((NF, NP, 128), jnp.float32),
    )(*args)


def _bounds(xc, attc, heads):
    def body(x_ref, a_ref, o_ref):
        cols = []
        if heads == 1:
            tot = jnp.zeros((BM,), jnp.float32)
            for f in range(NF):
                t = x_ref[f] * a_ref[f][None, :]
                tot = tot + jnp.sum(jnp.maximum(t, 0.2 * t), axis=-1)
            cols.append(tot)
        else:
            for h in range(heads):
                f, sl = (64 * h) // 128, (64 * h) % 128
                t = x_ref[f, :, sl:sl + 64] * a_ref[f, sl:sl + 64][None, :]
                cols.append(jnp.sum(jnp.maximum(t, 0.2 * t), axis=-1))
        stacked = jnp.stack(cols, axis=-1)
        o_ref[...] = jnp.concatenate(
            [stacked, jnp.zeros((BM, 128 - len(cols)), jnp.float32)],
            axis=-1)

    return pl.pallas_call(
        body, grid=(NP // BM,),
        in_specs=[pl.BlockSpec((NF, BM, 128), lambda i: (0, i, 0)),
                  pl.BlockSpec((NF, 128), lambda i: (0, 0))],
        out_specs=pl.BlockSpec((BM, 128), lambda i: (i, 0)),
        out_shape=jax.ShapeDtypeStruct((NP, 128), jnp.float32),
    )(xc, attc)


def _colmax(u):
    def body(u_ref, o_ref):
        i = pl.program_id(0)

        @pl.when(i == 0)
        def _():
            o_ref[...] = jnp.full((1, 128), -3e38, jnp.float32)

        o_ref[...] = jnp.maximum(o_ref[...],
                                 jnp.max(u_ref[...], axis=0, keepdims=True))

    return pl.pallas_call(
        body, grid=(NP // BM,),
        in_specs=[pl.BlockSpec((BM, 128), lambda i: (i, 0))],
        out_specs=pl.BlockSpec((1, 128), lambda i: (0, 0)),
        out_shape=jax.ShapeDtypeStruct((1, 128), jnp.float32),
    )(u)


def _s0_table(u2, mx):
    def body(u_ref, m_ref, o_ref):
        o_ref[...] = u_ref[...] + m_ref[...]

    return pl.pallas_call(
        body, grid=(NP // BM,),
        in_specs=[pl.BlockSpec((BM, 128), lambda i: (i, 0)),
                  pl.BlockSpec((1, 128), lambda i: (0, 0))],
        out_specs=pl.BlockSpec((BM, 128), lambda i: (i, 0)),
        out_shape=jax.ShapeDtypeStruct((NP, 128), jnp.float32),
    )(u2, mx)


def _lsr_table(d0p, heads, with_dinv):
    n_out = 2 if with_dinv else 1

    def body(d_ref, o_ref, *maybe_dinv):
        d0 = d_ref[0] + d_ref[1]
        ls = jnp.log(jnp.maximum(d0, 1e-35))
        den = d0 / jnp.maximum(d0, 1e-35)
        r = 1.0 / (den + 1e-16)
        colv = lax.broadcasted_iota(jnp.int32, (BM, 128), 1)
        r = jnp.where(colv < heads, r, 0.0)
        o_ref[...] = jnp.concatenate(
            [ls[:, :16], r[:, :16], jnp.zeros((BM, 96), jnp.float32)],
            axis=1)
        if with_dinv:
            dv = maybe_dinv[0]
            deg = d0[:, 10]
            dinv = 1.0 / jnp.sqrt(jnp.maximum(deg, 1.0))
            dv[...] = jnp.broadcast_to(dinv[:, None], (BM, 128))

    out_shape = [jax.ShapeDtypeStruct((NP, 128), jnp.float32)] * n_out
    res = pl.pallas_call(
        body, grid=(NP // BM,),
        in_specs=[pl.BlockSpec((2, BM, 128), lambda i: (0, i, 0))],
        out_specs=[pl.BlockSpec((BM, 128), lambda i: (i, 0))] * n_out,
        out_shape=out_shape,
    )(d0p)
    return res if with_dinv else (res[0],)


def _final_feat(aggc, dinv, b3c):
    def body(a_ref, d_ref, b_ref, o_ref):
        i = pl.program_id(1)
        r = jnp.maximum(a_ref[0] * d_ref[...] + b_ref[0], 0.0)
        rows = i * BM + lax.broadcasted_iota(jnp.int32, (BM, 128), 0)
        o_ref[0] = jnp.where(rows < N, r, 0.0)

    return pl.pallas_call(
        body, grid=(NF, NP // BM),
        in_specs=[pl.BlockSpec((1, BM, 128), lambda f, i: (f, i, 0)),
                  pl.BlockSpec((BM, 128), lambda f, i: (i, 0)),
                  pl.BlockSpec((1, 1, 128), lambda f, i: (f, 0, 0))],
        out_specs=pl.BlockSpec((1, BM, 128), lambda f, i: (f, i, 0)),
        out_shape=jax.ShapeDtypeStruct((NF, NP, 128), jnp.float32),
    )(aggc, dinv, b3c.reshape(NF, 1, 128))


def _graph_tables(batch2d):
    def body(b_ref, c_ref, s_ref):
        i = pl.program_id(0)

        @pl.when(i == 0)
        def _():
            c_ref[...] = jnp.zeros((1, NG), jnp.int32)
            s_ref[...] = jnp.zeros((1, NG), jnp.int32)

        b = b_ref[...]
        g = lax.broadcasted_iota(jnp.int32, (BM, NG), 1)
        c_ref[...] += jnp.sum((b == g).astype(jnp.int32), axis=0,
                              keepdims=True)
        s_ref[...] += jnp.sum((b < g).astype(jnp.int32), axis=0,
                              keepdims=True)

    return pl.pallas_call(
        body, grid=(NP // BM,),
        in_specs=[pl.BlockSpec((BM, 1), lambda i: (i, 0))],
        out_specs=[pl.BlockSpec((1, NG), lambda i: (0, 0))] * 2,
        out_shape=[jax.ShapeDtypeStruct((1, NG), jnp.int32)] * 2,
    )(batch2d)


def _sc_wid():
    return lax.axis_index("s") * 2 + lax.axis_index("c")


def _edge_logits(heads, with_deg):
    CA = 24
    PT = EP // 32
    nch = PT // CA
    ch_head = HID // heads

    scratch = [pltpu.VMEM((HID,), jnp.float32)]
    scratch += [pltpu.VMEM((CA, 128), jnp.float32) for _ in range(10)]
    scratch += [pltpu.VMEM((CA, 128), jnp.float32)]
    scratch += [pltpu.VMEM((CA,), jnp.int32),
                pltpu.VMEM((CA,), jnp.int32),
                pltpu.VMEM((CA * 16,), jnp.float32),
                pltpu.VMEM((CA, 128), jnp.float32),
                pltpu.VMEM((16, 128), jnp.float32),
                pltpu.VMEM_SHARED((NP, 128), jnp.float32),
                pltpu.SemaphoreType.DMA]

    @functools.partial(
        pl.kernel,
        out_type=[jax.ShapeDtypeStruct((EP * 16,), jnp.float32),
                  jax.ShapeDtypeStruct((2, NP, 128), jnp.float32)],
        mesh=_mesh(),
        compiler_params=pltpu.CompilerParams(needs_layout_passes=False),
        scratch_types=scratch)
    def k(xlc, xrc, s0t, att_h, srca, dsta, l_out, d0_out,
          attv, xb0, xb1, xb2, xb3, xb4, rb0, rb1, rb2, rb3, rb4, s0b,
          idxs, idxd, lbuf, pbuf, zbuf, d0s, sem):
        cc = lax.axis_index("c")
        wid = _sc_wid()
        xlb = [xb0, xb1, xb2, xb3, xb4]
        xrb = [rb0, rb1, rb2, rb3, rb4]
        lane = lax.iota(jnp.int32, 16)

        pltpu.sync_copy(att_h, attv)
        for i in range(16):
            for q in range(8):
                zbuf[i, pl.ds(16 * q, 16)] = jnp.zeros((16,), jnp.float32)
        for b in range(40):
            r0 = lax.axis_index("s") * 640 + b * 16
            pltpu.sync_copy(zbuf, d0s.at[pl.ds(r0, 16), :])
        for i in range(CA):
            for q in range(1, 8):
                pbuf[i, pl.ds(16 * q, 16)] = jnp.zeros((16,), jnp.float32)
        plsc.subcore_barrier()

        tile_base = wid * PT

        def chunk(ci, _):
            base = pl.multiple_of(tile_base + ci * CA, 8)
            di1 = pltpu.async_copy(srca.at[pl.ds(base, CA)], idxs, sem)
            di2 = pltpu.async_copy(dsta.at[pl.ds(base, CA)], idxd, sem)
            di1.wait()
            di2.wait()
            descs = []
            for f in range(NF):
                descs.append(pltpu.async_copy(xlc.at[f].at[idxs],
                                              xlb[f], sem))
                descs.append(pltpu.async_copy(xrc.at[f].at[idxd],
                                              xrb[f], sem))
            descs.append(pltpu.async_copy(s0t.at[idxd], s0b, sem))
            for d in descs:
                d.wait()

            def edge(i, _):
                tots = []
                for h in range(heads):
                    acc = None
                    for q in range(ch_head // 16):
                        g = h * ch_head + q * 16
                        f, off = g // 128, g % 128
                        a = xlb[f][i, pl.ds(off, 16)]
                        b = xrb[f][i, pl.ds(off, 16)]
                        z = a + b
                        w = jnp.maximum(z, 0.2 * z) * attv[pl.ds(g, 16)]
                        acc = w if acc is None else acc + w
                    tots.append(jnp.sum(acc))
                lrow = jnp.zeros((16,), jnp.float32)
                for h in range(heads):
                    lrow = jnp.where(lane == h,
                                     jnp.full((16,), tots[h], jnp.float32),
                                     lrow)
                s0r = s0b[i, pl.ds(0, 16)]
                hm = lane < heads
                ls = jnp.where(hm, lrow - s0r, 0.0)
                p = jnp.where(hm, jnp.exp(ls), 0.0)
                if with_deg:
                    p = p + jnp.where(lane == 10, 1.0, 0.0)
                lbuf[pl.ds(i * 16, 16)] = ls
                pbuf[i, pl.ds(0, 16)] = p
                return 0

            lax.fori_loop(0, CA, edge, 0)
            pltpu.sync_copy(lbuf, l_out.at[pl.ds(base * 16, CA * 16)])
            pltpu.sync_copy(pbuf, d0s.at[idxd], add=True)
            return 0

        lax.fori_loop(0, nch, chunk, 0)
        plsc.subcore_barrier()
        r0 = lax.axis_index("s") * 640
        pltpu.sync_copy(d0s.at[pl.ds(r0, 640), :],
                        d0_out.at[cc].at[pl.ds(r0, 640), :])

    return k


def _alpha_pass():
    CB = 64
    PT = EP // 32
    nch = PT // CB

    @functools.partial(
        pl.kernel,
        out_type=jax.ShapeDtypeStruct((EP * 16,), jnp.float32),
        mesh=_mesh(),
        compiler_params=pltpu.CompilerParams(needs_layout_passes=False),
        scratch_types=[pltpu.VMEM((CB * 16,), jnp.float32),
                       pltpu.VMEM((CB, 128), jnp.float32),
                       pltpu.VMEM((CB,), jnp.int32),
                       pltpu.VMEM((CB * 16,), jnp.float32),
                       pltpu.SemaphoreType.DMA])
    def k(l_in, lsr, dsta, a_out, lb, lsrb, idxd, ab, sem):
        wid = _sc_wid()
        tile_base = wid * PT

        def chunk(ci, _):
            base = pl.multiple_of(tile_base + ci * CB, 8)
            pltpu.sync_copy(dsta.at[pl.ds(base, CB)], idxd)
            d1 = pltpu.async_copy(l_in.at[pl.ds(base * 16, CB * 16)], lb,
                                  sem)
            d2 = pltpu.async_copy(lsr.at[idxd], lsrb, sem)
            d1.wait()
            d2.wait()

            def edge(i, _):
                ls = lsrb[i, pl.ds(0, 16)]
                r = lsrb[i, pl.ds(16, 16)]
                ab[pl.ds(i * 16, 16)] = (
                    jnp.exp(lb[pl.ds(i * 16, 16)] - ls) * r)
                return 0

            lax.fori_loop(0, CB, edge, 0)
            pltpu.sync_copy(ab, a_out.at[pl.ds(base * 16, CB * 16)])
            return 0

        lax.fori_loop(0, nch, chunk, 0)

    return k


def _aggregate(heads):
    CC = 96
    SB = 8
    PT = EP // 16
    ntch = PT // CC
    nsb = ntch // SB
    has_alpha = heads > 0

    scratch = [pltpu.VMEM((CC, 128), jnp.float32),
               pltpu.VMEM((CC, 128), jnp.float32),
               pltpu.VMEM((SB * CC * 16,), jnp.float32),
               pltpu.VMEM((SB, CC), jnp.int32),
               pltpu.VMEM((SB, CC), jnp.int32),
               pltpu.VMEM((16, 128), jnp.float32),
               pltpu.VMEM_SHARED((NP, 128), jnp.float32),
               pltpu.SemaphoreType.DMA,
               pltpu.SemaphoreType.DMA]

    @functools.partial(
        pl.kernel,
        out_type=jax.ShapeDtypeStruct((NF, NP, 128), jnp.float32),
        mesh=_mesh(),
        compiler_params=pltpu.CompilerParams(needs_layout_passes=False),
        scratch_types=scratch)
    def k(xlc, alpha, src_r, dst_r, agg,
          xb0, xb1, alb, isb, idb, zbuf, accs, semg, sems):
        cc = lax.axis_index("c")
        sid = lax.axis_index("s")
        xbs = [xb0, xb1]

        for i in range(16):
            for q in range(8):
                zbuf[i, pl.ds(16 * q, 16)] = jnp.zeros((16,), jnp.float32)

        for f in range(NF):
            @pl.when(cc == _OWNER[f])
            def _(f=f):
                for b in range(40):
                    r0 = sid * 640 + b * 16
                    pltpu.sync_copy(zbuf, accs.at[pl.ds(r0, 16), :])
                plsc.subcore_barrier()

                def sblock(sb, _):
                    row0 = pl.multiple_of(sid * ntch + sb * SB, 8)
                    pltpu.sync_copy(src_r.at[pl.ds(row0, SB), :], isb)
                    pltpu.sync_copy(dst_r.at[pl.ds(row0, SB), :], idb)
                    if has_alpha:
                        eb = pl.multiple_of(row0 * CC * 16, 8)
                        pltpu.sync_copy(alpha.at[pl.ds(eb, SB * CC * 16)],
                                        alb)

                    def g_desc(kk, buf):
                        return pltpu.make_async_copy(
                            xlc.at[f].at[isb.at[kk]], buf, semg)

                    def s_desc(kk, buf):
                        return pltpu.make_async_copy(
                            buf, accs.at[idb.at[kk]], sems)

                    g_desc(0, xbs[0]).start()
                    last_sc = [None, None]
                    for kk in range(SB):
                        p = kk % 2
                        if kk + 1 < SB:
                            q = (kk + 1) % 2
                            if last_sc[q] is not None:
                                s_desc(last_sc[q], xbs[q]).wait()
                            g_desc(kk + 1, xbs[q]).start()
                        g_desc(kk, xbs[p]).wait()
                        if has_alpha:
                            xb = xbs[p]

                            def edge(i, _, kk=kk, xb=xb):
                                arow = alb[pl.ds((kk * CC + i) * 16, 16)]
                                if heads == 1:
                                    a0 = arow[0]
                                    a1 = a0
                                else:
                                    a0 = arow[2 * f]
                                    a1 = arow[2 * f + 1]
                                av0 = jnp.full((16,), a0, jnp.float32)
                                av1 = jnp.full((16,), a1, jnp.float32)
                                for q2 in range(8):
                                    av = av0 if q2 < 4 else av1
                                    xb[i, pl.ds(16 * q2, 16)] = (
                                        xb[i, pl.ds(16 * q2, 16)] * av)
                                return 0

                            lax.fori_loop(0, CC, edge, 0)
                        pltpu.async_copy(xbs[p], accs.at[idb.at[kk]],
                                         sems, add=True)
                        last_sc[p] = kk
                    for q in (0, 1):
                        if last_sc[q] is not None:
                            s_desc(last_sc[q], xbs[q]).wait()
                    return 0

                lax.fori_loop(0, nsb, sblock, 0)
                plsc.subcore_barrier()
                r0 = sid * 640
                pltpu.sync_copy(accs.at[pl.ds(r0, 640), :],
                                agg.at[f].at[pl.ds(r0, 640), :])
                plsc.subcore_barrier()

    return k


def _pooling():
    @functools.partial(
        pl.kernel,
        out_type=jax.ShapeDtypeStruct((NG, 2 * HID), jnp.float32),
        mesh=_mesh(),
        compiler_params=pltpu.CompilerParams(needs_layout_passes=False),
        scratch_types=[pltpu.VMEM((272,), jnp.int32),
                       pltpu.VMEM((272,), jnp.int32),
                       pltpu.VMEM((8, 128), jnp.float32),
                       pltpu.VMEM((2 * HID,), jnp.float32),
                       pltpu.SemaphoreType.DMA])
    def k(hf, starts_h, cnts_h, out, sv, cv, buf, rowbuf, sem):
        wid = _sc_wid()
        pltpu.sync_copy(starts_h, sv)
        pltpu.sync_copy(cnts_h, cv)
        srow = sv[pl.ds(8 * wid, 16)]
        crow = cv[pl.ds(8 * wid, 16)]
        for j in range(8):
            st = srow[j]
            cn = crow[j]
            st8 = pl.multiple_of(st & (-8), 8)
            dlt = st - st8
            nb = (cn + dlt + 7) >> 3
            cnf = jnp.maximum(cn.astype(jnp.float32), 1.0)
            invv = 1.0 / jnp.full((16,), cnf, jnp.float32)
            nz = cn > 0
            for f in range(NF):
                def blk(b, carry):
                    pltpu.async_copy(
                        hf.at[f].at[pl.ds(st8 + 8 * b, 8), :], buf,
                        sem).wait()
                    out_c = []
                    for q in range(8):
                        mx, sm = carry[2 * q], carry[2 * q + 1]
                        for r in range(8):
                            idx = 8 * b + r
                            valid = (idx >= dlt) & (idx < dlt + cn)
                            vm = jnp.full((16,), valid, jnp.bool_)
                            v = buf[r, pl.ds(16 * q, 16)]
                            mx = jnp.maximum(mx, jnp.where(vm, v, -3e38))
                            sm = sm + jnp.where(vm, v, 0.0)
                        out_c += [mx, sm]
                    return tuple(out_c)

                init = []
                for q in range(8):
                    init += [jnp.full((16,), -3e38, jnp.float32),
                             jnp.zeros((16,), jnp.float32)]
                res = lax.fori_loop(0, nb, blk, tuple(init))
                nzv = jnp.full((16,), nz, jnp.bool_)
                for q in range(8):
                    gmax = jnp.where(nzv, res[2 * q], 0.0)
                    gmax = jnp.where(nzv, gmax, 0.0)
                    gmean = res[2 * q + 1] * invv
                    rowbuf[pl.ds(128 * f + 16 * q, 16)] = gmax
                    rowbuf[pl.ds(HID + 128 * f + 16 * q, 16)] = gmean
            pltpu.sync_copy(rowbuf, out.at[8 * wid + j])

    return k


def _gatv2_layer(xc, src, dst, wl, wr, attf, bias, heads, with_deg,
                 in_bias=None, in_act=None):
    kc = xc.shape[0]
    wl_r = wl.reshape(kc, 128, NF, 128).transpose(0, 2, 1, 3)
    wr_r = wr.reshape(kc, 128, NF, 128).transpose(0, 2, 1, 3)
    xlc = _mm_chunked(xc, wl_r, bias=in_bias, act=in_act)
    xrc = _mm_chunked(xc, wr_r, bias=in_bias, act=in_act)
    attc = attf.reshape(NF, 128)
    u1 = _bounds(xlc, attc, heads)
    u2 = _bounds(xrc, attc, heads)
    mx = _colmax(u1)
    s0 = _s0_table(u2, mx)
    l_e, d0p = _edge_logits(heads, with_deg)(
        xlc, xrc, s0, attf, src, dst)
    tabs = _lsr_table(d0p, heads, with_deg)
    lsr = tabs[0]
    dinv = tabs[1] if with_deg else None
    alpha = _alpha_pass()(l_e, lsr, dst)
    agg = _aggregate(heads)(xlc, alpha, src.reshape(-1, 96),
                            dst.reshape(-1, 96))
    return agg, dinv


def kernel(x, edge_index, batch, Wl1, Wr1, att1, b1, Wl2, Wr2, att2, b2,
           W3, b3):
    f32 = jnp.float32
    x = x.astype(f32)
    xp = jnp.pad(x, ((0, NP - N), (0, 0))).reshape(1, NP, 128)
    sl = jnp.arange(N, dtype=jnp.int32)
    src = jnp.concatenate([edge_index[0].astype(jnp.int32), sl,
                           jnp.full((EP - E0 - N,), N, jnp.int32)])
    dst = jnp.concatenate([edge_index[1].astype(jnp.int32), sl,
                           jnp.full((EP - E0 - N,), N, jnp.int32)])

    agg1, dinv = _gatv2_layer(xp, src, dst, Wl1, Wr1, att1.reshape(-1),
                              b1, 10, True)
    b1c = b1.reshape(NF, 128)
    agg2, _ = _gatv2_layer(agg1, src, dst, Wl2, Wr2, att2.reshape(-1),
                           b2, 1, False, in_bias=b1c, in_act="elu")
    b2c = b2.reshape(NF, 128)
    w3_r = W3.reshape(NF, 128, NF, 128).transpose(0, 2, 1, 3)
    hws = _mm_chunked(agg2, w3_r, bias=b2c, out_scale=dinv)
    dummy_a = jnp.zeros((128,), jnp.float32)
    aggg = _aggregate(0)(hws, dummy_a, src.reshape(-1, 96),
                         dst.reshape(-1, 96))
    hf = _final_feat(aggg, dinv, b3.reshape(NF, 128))

    batchp = jnp.pad(batch.astype(jnp.int32), (0, NP - N),
                     constant_values=NG)
    cnt, starts = _graph_tables(batchp.reshape(NP, 1))
    starts_p = jnp.pad(starts.reshape(NG), (0, 16))
    cnts_p = jnp.pad(cnt.reshape(NG), (0, 16))
    return _pooling()(hf, starts_p, cnts_p)


# speedup vs baseline: 6.9517x; 1.0420x over previous
import functools

import jax
import jax.numpy as jnp
from jax import lax
from jax.experimental import pallas as pl
from jax.experimental.pallas import tpu as pltpu
from jax.experimental.pallas import tpu_sc as plsc

N = 10000
NP = 10240
E0 = 320000
EP = 331776
HID = 640
NF = 5
NG = 256
BM = 512
def _mesh():
    return plsc.VectorSubcoreMesh(core_axis_name="c", subcore_axis_name="s")
_OWNER = (0, 0, 1, 1, 0)


def _mm_chunked(xc, wr, bias=None, act=None, out_scale=None, x4=None):
    kc = xc.shape[0]
    grid = (NF, NP // BM, kc)

    def body(*refs):
        refs = list(refs)
        x_ref = refs.pop(0)
        w_ref = refs.pop(0)
        e_ref = refs.pop(0) if x4 is not None else None
        b_ref = refs.pop(0) if bias is not None else None
        s_ref = refs.pop(0) if out_scale is not None else None
        o_ref = refs.pop(0)
        i = pl.program_id(1)
        ki = pl.program_id(2)
        x = x_ref[0]
        if e_ref is not None:
            x = x + jnp.where(ki == 4, 1.0, 0.0) * e_ref[...]
        if b_ref is not None:
            x = x + b_ref[0]
        if act == "elu":
            x = jnp.where(x > 0, x, jnp.exp(x) - 1.0)
        acc = jnp.dot(x, w_ref[0, 0],
                      preferred_element_type=jnp.float32)

        @pl.when(ki == 0)
        def _():
            o_ref[0] = jnp.zeros_like(o_ref[0])

        o_ref[0] += acc

        @pl.when(ki == kc - 1)
        def _():
            r = o_ref[0]
            if s_ref is not None:
                r = r * s_ref[...]
            rows = i * BM + lax.broadcasted_iota(jnp.int32, (BM, 128), 0)
            o_ref[0] = jnp.where(rows < N, r, 0.0)

    specs = [
        pl.BlockSpec((1, BM, 128), lambda j, i, ki: (ki, i, 0)),
        pl.BlockSpec((1, 1, 128, 128), lambda j, i, ki: (ki, j, 0, 0)),
    ]
    args = [xc, wr]
    if x4 is not None:
        specs.append(pl.BlockSpec((BM, 128), lambda j, i, ki: (i, 0)))
        args.append(x4)
    if bias is not None:
        specs.append(pl.BlockSpec((1, 1, 128), lambda j, i, ki: (ki, 0, 0)))
        args.append(bias.reshape(-1, 1, 128))
    if out_scale is not None:
        specs.append(pl.BlockSpec((BM, 128), lambda j, i, ki: (i, 0)))
        args.append(out_scale)
    return pl.pallas_call(
        body, grid=grid, in_specs=specs,
        out_specs=pl.BlockSpec((1, BM, 128), lambda j, i, ki: (j, i, 0)),
        out_shape=jax.ShapeDtypeStruct((NF, NP, 128), jnp.float32),
    )(*args)


def _bounds(xc, attc, heads):
    def body(x_ref, a_ref, o_ref):
        cols = []
        if heads == 1:
            tot = jnp.zeros((BM,), jnp.float32)
            for f in range(NF):
                t = x_ref[f] * a_ref[f][None, :]
                tot = tot + jnp.sum(jnp.maximum(t, 0.2 * t), axis=-1)
            cols.append(tot)
        else:
            for h in range(heads):
                f, sl = (64 * h) // 128, (64 * h) % 128
                t = x_ref[f, :, sl:sl + 64] * a_ref[f, sl:sl + 64][None, :]
                cols.append(jnp.sum(jnp.maximum(t, 0.2 * t), axis=-1))
        stacked = jnp.stack(cols, axis=-1)
        o_ref[...] = jnp.concatenate(
            [stacked, jnp.zeros((BM, 128 - len(cols)), jnp.float32)],
            axis=-1)

    return pl.pallas_call(
        body, grid=(NP // BM,),
        in_specs=[pl.BlockSpec((NF, BM, 128), lambda i: (0, i, 0)),
                  pl.BlockSpec((NF, 128), lambda i: (0, 0))],
        out_specs=pl.BlockSpec((BM, 128), lambda i: (i, 0)),
        out_shape=jax.ShapeDtypeStruct((NP, 128), jnp.float32),
    )(xc, attc)


def _colmax(u):
    def body(u_ref, o_ref):
        i = pl.program_id(0)

        @pl.when(i == 0)
        def _():
            o_ref[...] = jnp.full((1, 128), -3e38, jnp.float32)

        o_ref[...] = jnp.maximum(o_ref[...],
                                 jnp.max(u_ref[...], axis=0, keepdims=True))

    return pl.pallas_call(
        body, grid=(NP // BM,),
        in_specs=[pl.BlockSpec((BM, 128), lambda i: (i, 0))],
        out_specs=pl.BlockSpec((1, 128), lambda i: (0, 0)),
        out_shape=jax.ShapeDtypeStruct((1, 128), jnp.float32),
    )(u)


def _s0_table(u2, mx):
    def body(u_ref, m_ref, o_ref):
        o_ref[...] = u_ref[...] + m_ref[...]

    return pl.pallas_call(
        body, grid=(NP // BM,),
        in_specs=[pl.BlockSpec((BM, 128), lambda i: (i, 0)),
                  pl.BlockSpec((1, 128), lambda i: (0, 0))],
        out_specs=pl.BlockSpec((BM, 128), lambda i: (i, 0)),
        out_shape=jax.ShapeDtypeStruct((NP, 128), jnp.float32),
    )(u2, mx)


def _lsr_table(d0p, heads, with_dinv):
    n_out = 2 if with_dinv else 1

    def body(d_ref, o_ref, *maybe_dinv):
        d0 = d_ref[0] + d_ref[1]
        ls = jnp.log(jnp.maximum(d0, 1e-35))
        den = d0 / jnp.maximum(d0, 1e-35)
        r = 1.0 / (den + 1e-16)
        colv = lax.broadcasted_iota(jnp.int32, (BM, 128), 1)
        r = jnp.where(colv < heads, r, 0.0)
        o_ref[...] = jnp.concatenate(
            [ls[:, :16], r[:, :16], jnp.zeros((BM, 96), jnp.float32)],
            axis=1)
        if with_dinv:
            dv = maybe_dinv[0]
            deg = d0[:, 10]
            dinv = 1.0 / jnp.sqrt(jnp.maximum(deg, 1.0))
            dv[...] = jnp.broadcast_to(dinv[:, None], (BM, 128))

    out_shape = [jax.ShapeDtypeStruct((NP, 128), jnp.float32)] * n_out
    res = pl.pallas_call(
        body, grid=(NP // BM,),
        in_specs=[pl.BlockSpec((2, BM, 128), lambda i: (0, i, 0))],
        out_specs=[pl.BlockSpec((BM, 128), lambda i: (i, 0))] * n_out,
        out_shape=out_shape,
    )(d0p)
    return res if with_dinv else (res[0],)


def _final_feat(aggc, agg4b, dinv, b3c):
    def body(a_ref, e_ref, d_ref, b_ref, o_ref):
        f = pl.program_id(0)
        i = pl.program_id(1)
        a = a_ref[0] + jnp.where(f == 4, 1.0, 0.0) * e_ref[...]
        r = jnp.maximum(a * d_ref[...] + b_ref[0], 0.0)
        rows = i * BM + lax.broadcasted_iota(jnp.int32, (BM, 128), 0)
        o_ref[0] = jnp.where(rows < N, r, 0.0)

    return pl.pallas_call(
        body, grid=(NF, NP // BM),
        in_specs=[pl.BlockSpec((1, BM, 128), lambda f, i: (f, i, 0)),
                  pl.BlockSpec((BM, 128), lambda f, i: (i, 0)),
                  pl.BlockSpec((BM, 128), lambda f, i: (i, 0)),
                  pl.BlockSpec((1, 1, 128), lambda f, i: (f, 0, 0))],
        out_specs=pl.BlockSpec((1, BM, 128), lambda f, i: (f, i, 0)),
        out_shape=jax.ShapeDtypeStruct((NF, NP, 128), jnp.float32),
    )(aggc, agg4b, dinv, b3c.reshape(NF, 1, 128))


def _graph_tables(batch2d):
    def body(b_ref, c_ref, s_ref):
        i = pl.program_id(0)

        @pl.when(i == 0)
        def _():
            c_ref[...] = jnp.zeros((1, NG), jnp.int32)
            s_ref[...] = jnp.zeros((1, NG), jnp.int32)

        b = b_ref[...]
        g = lax.broadcasted_iota(jnp.int32, (BM, NG), 1)
        c_ref[...] += jnp.sum((b == g).astype(jnp.int32), axis=0,
                              keepdims=True)
        s_ref[...] += jnp.sum((b < g).astype(jnp.int32), axis=0,
                              keepdims=True)

    return pl.pallas_call(
        body, grid=(NP // BM,),
        in_specs=[pl.BlockSpec((BM, 1), lambda i: (i, 0))],
        out_specs=[pl.BlockSpec((1, NG), lambda i: (0, 0))] * 2,
        out_shape=[jax.ShapeDtypeStruct((1, NG), jnp.int32)] * 2,
    )(batch2d)


def _sc_wid():
    return lax.axis_index("s") * 2 + lax.axis_index("c")


def _edge_logits(heads, with_deg):
    CA = 24
    PT = EP // 32
    nch = PT // CA
    ch_head = HID // heads

    scratch = [pltpu.VMEM((HID,), jnp.float32)]
    scratch += [pltpu.VMEM((CA, 128), jnp.float32) for _ in range(10)]
    scratch += [pltpu.VMEM((CA, 128), jnp.float32)]
    scratch += [pltpu.VMEM((CA,), jnp.int32),
                pltpu.VMEM((CA,), jnp.int32),
                pltpu.VMEM((CA * 16,), jnp.float32),
                pltpu.VMEM((CA, 128), jnp.float32),
                pltpu.VMEM((16, 128), jnp.float32),
                pltpu.VMEM_SHARED((NP, 128), jnp.float32),
                pltpu.SemaphoreType.DMA]

    @functools.partial(
        pl.kernel,
        out_type=[jax.ShapeDtypeStruct((EP * 16,), jnp.float32),
                  jax.ShapeDtypeStruct((2, NP, 128), jnp.float32)],
        mesh=_mesh(),
        compiler_params=pltpu.CompilerParams(needs_layout_passes=False),
        scratch_types=scratch)
    def k(xlc, xrc, s0t, att_h, srca, dsta, l_out, d0_out,
          attv, xb0, xb1, xb2, xb3, xb4, rb0, rb1, rb2, rb3, rb4, s0b,
          idxs, idxd, lbuf, pbuf, zbuf, d0s, sem):
        cc = lax.axis_index("c")
        wid = _sc_wid()
        xlb = [xb0, xb1, xb2, xb3, xb4]
        xrb = [rb0, rb1, rb2, rb3, rb4]
        lane = lax.iota(jnp.int32, 16)

        pltpu.sync_copy(att_h, attv)
        for i in range(16):
            for q in range(8):
                zbuf[i, pl.ds(16 * q, 16)] = jnp.zeros((16,), jnp.float32)
        for b in range(40):
            r0 = lax.axis_index("s") * 640 + b * 16
            pltpu.sync_copy(zbuf, d0s.at[pl.ds(r0, 16), :])
        for i in range(CA):
            for q in range(1, 8):
                pbuf[i, pl.ds(16 * q, 16)] = jnp.zeros((16,), jnp.float32)
        plsc.subcore_barrier()

        tile_base = wid * PT

        def chunk(ci, _):
            base = pl.multiple_of(tile_base + ci * CA, 8)
            di1 = pltpu.async_copy(srca.at[pl.ds(base, CA)], idxs, sem)
            di2 = pltpu.async_copy(dsta.at[pl.ds(base, CA)], idxd, sem)
            di1.wait()
            di2.wait()
            descs = []
            for f in range(NF):
                descs.append(pltpu.async_copy(xlc.at[f].at[idxs],
                                              xlb[f], sem))
                descs.append(pltpu.async_copy(xrc.at[f].at[idxd],
                                              xrb[f], sem))
            descs.append(pltpu.async_copy(s0t.at[idxd], s0b, sem))
            for d in descs:
                d.wait()

            def edge(i, _):
                tots = []
                for h in range(heads):
                    acc = None
                    for q in range(ch_head // 16):
                        g = h * ch_head + q * 16
                        f, off = g // 128, g % 128
                        a = xlb[f][i, pl.ds(off, 16)]
                        b = xrb[f][i, pl.ds(off, 16)]
                        z = a + b
                        w = jnp.maximum(z, 0.2 * z) * attv[pl.ds(g, 16)]
                        acc = w if acc is None else acc + w
                    tots.append(jnp.sum(acc))
                lrow = jnp.zeros((16,), jnp.float32)
                for h in range(heads):
                    lrow = jnp.where(lane == h,
                                     jnp.full((16,), tots[h], jnp.float32),
                                     lrow)
                s0r = s0b[i, pl.ds(0, 16)]
                hm = lane < heads
                ls = jnp.where(hm, lrow - s0r, 0.0)
                p = jnp.where(hm, jnp.exp(ls), 0.0)
                if with_deg:
                    p = p + jnp.where(lane == 10, 1.0, 0.0)
                lbuf[pl.ds(i * 16, 16)] = ls
                pbuf[i, pl.ds(0, 16)] = p
                return 0

            lax.fori_loop(0, CA, edge, 0)
            pltpu.sync_copy(lbuf, l_out.at[pl.ds(base * 16, CA * 16)])
            pltpu.sync_copy(pbuf, d0s.at[idxd], add=True)
            return 0

        lax.fori_loop(0, nch, chunk, 0)
        plsc.subcore_barrier()
        r0 = lax.axis_index("s") * 640
        pltpu.sync_copy(d0s.at[pl.ds(r0, 640), :],
                        d0_out.at[cc].at[pl.ds(r0, 640), :])

    return k


def _alpha_pass():
    CB = 64
    PT = EP // 32
    nch = PT // CB

    @functools.partial(
        pl.kernel,
        out_type=jax.ShapeDtypeStruct((EP * 16,), jnp.float32),
        mesh=_mesh(),
        compiler_params=pltpu.CompilerParams(needs_layout_passes=False),
        scratch_types=[pltpu.VMEM((CB * 16,), jnp.float32),
                       pltpu.VMEM((CB, 128), jnp.float32),
                       pltpu.VMEM((CB,), jnp.int32),
                       pltpu.VMEM((CB * 16,), jnp.float32),
                       pltpu.SemaphoreType.DMA])
    def k(l_in, lsr, dsta, a_out, lb, lsrb, idxd, ab, sem):
        wid = _sc_wid()
        tile_base = wid * PT

        def chunk(ci, _):
            base = pl.multiple_of(tile_base + ci * CB, 8)
            pltpu.sync_copy(dsta.at[pl.ds(base, CB)], idxd)
            d1 = pltpu.async_copy(l_in.at[pl.ds(base * 16, CB * 16)], lb,
                                  sem)
            d2 = pltpu.async_copy(lsr.at[idxd], lsrb, sem)
            d1.wait()
            d2.wait()

            def edge(i, _):
                ls = lsrb[i, pl.ds(0, 16)]
                r = lsrb[i, pl.ds(16, 16)]
                ab[pl.ds(i * 16, 16)] = (
                    jnp.exp(lb[pl.ds(i * 16, 16)] - ls) * r)
                return 0

            lax.fori_loop(0, CB, edge, 0)
            pltpu.sync_copy(ab, a_out.at[pl.ds(base * 16, CB * 16)])
            return 0

        lax.fori_loop(0, nch, chunk, 0)

    return k


def _aggregate(heads):
    CC = 96
    SB = 8
    PT = EP // 16
    ntch = PT // CC
    nsb = ntch // SB
    has_alpha = heads > 0

    scratch = [pltpu.VMEM((CC, 128), jnp.float32),
               pltpu.VMEM((CC, 128), jnp.float32),
               pltpu.VMEM((SB * CC * 16,), jnp.float32),
               pltpu.VMEM((SB, CC), jnp.int32),
               pltpu.VMEM((SB, CC), jnp.int32),
               pltpu.VMEM((16, 128), jnp.float32),
               pltpu.VMEM_SHARED((NP, 128), jnp.float32),
               pltpu.SemaphoreType.DMA,
               pltpu.SemaphoreType.DMA]

    @functools.partial(
        pl.kernel,
        out_type=[jax.ShapeDtypeStruct((NF, NP, 128), jnp.float32),
                  jax.ShapeDtypeStruct((NP, 128), jnp.float32)],
        mesh=_mesh(),
        compiler_params=pltpu.CompilerParams(needs_layout_passes=False),
        scratch_types=scratch)
    def k(xlc, alpha, src_r, dst_r, agg, agg4b,
          xb0, xb1, alb, isb, idb, zbuf, accs, semg, sems):
        cc = lax.axis_index("c")
        sid = lax.axis_index("s")
        xbs = [xb0, xb1]

        for i in range(16):
            for q in range(8):
                zbuf[i, pl.ds(16 * q, 16)] = jnp.zeros((16,), jnp.float32)

        for f in range(NF):
            @pl.when((cc == _OWNER[f]) | (f == 4))
            def _(f=f):
                for b in range(40):
                    r0 = sid * 640 + b * 16
                    pltpu.sync_copy(zbuf, accs.at[pl.ds(r0, 16), :])
                plsc.subcore_barrier()

                def sblock(sb, _):
                    row0 = pl.multiple_of(sid * ntch + sb * SB, 8)
                    pltpu.sync_copy(src_r.at[pl.ds(row0, SB), :], isb)
                    pltpu.sync_copy(dst_r.at[pl.ds(row0, SB), :], idb)
                    if has_alpha:
                        eb = pl.multiple_of(row0 * CC * 16, 8)
                        pltpu.sync_copy(alpha.at[pl.ds(eb, SB * CC * 16)],
                                        alb)

                    def g_desc(kk, buf):
                        return pltpu.make_async_copy(
                            xlc.at[f].at[isb.at[kk]], buf, semg)

                    def s_desc(kk, buf):
                        return pltpu.make_async_copy(
                            buf, accs.at[idb.at[kk]], sems)

                    g_desc(0, xbs[0]).start()
                    last_sc = [None, None]
                    for kk in range(SB):
                        p = kk % 2
                        if kk + 1 < SB:
                            q = (kk + 1) % 2
                            if last_sc[q] is not None:
                                s_desc(last_sc[q], xbs[q]).wait()
                            g_desc(kk + 1, xbs[q]).start()
                        g_desc(kk, xbs[p]).wait()
                        if has_alpha:
                            xb = xbs[p]

                            def edge(i, _, kk=kk, xb=xb):
                                arow = alb[pl.ds((kk * CC + i) * 16, 16)]
                                if heads == 1:
                                    a0 = arow[0]
                                    a1 = a0
                                else:
                                    a0 = arow[2 * f]
                                    a1 = arow[2 * f + 1]
                                av0 = jnp.full((16,), a0, jnp.float32)
                                av1 = jnp.full((16,), a1, jnp.float32)
                                for q2 in range(8):
                                    av = av0 if q2 < 4 else av1
                                    xb[i, pl.ds(16 * q2, 16)] = (
                                        xb[i, pl.ds(16 * q2, 16)] * av)
                                return 0

                            lax.fori_loop(0, CC, edge, 0)
                        pltpu.async_copy(xbs[p], accs.at[idb.at[kk]],
                                         sems, add=True)
                        last_sc[p] = kk
                    for q in (0, 1):
                        if last_sc[q] is not None:
                            s_desc(last_sc[q], xbs[q]).wait()
                    return 0

                if f == 4:
                    lo = jnp.where(cc == 0, 0, 13)
                    hi = jnp.where(cc == 0, 13, nsb)
                    lax.fori_loop(lo, hi, sblock, 0)
                else:
                    lax.fori_loop(0, nsb, sblock, 0)
                plsc.subcore_barrier()
                r0 = sid * 640
                if f == 4:
                    @pl.when(cc == 0)
                    def _():
                        pltpu.sync_copy(accs.at[pl.ds(r0, 640), :],
                                        agg.at[f].at[pl.ds(r0, 640), :])

                    @pl.when(cc == 1)
                    def _():
                        pltpu.sync_copy(accs.at[pl.ds(r0, 640), :],
                                        agg4b.at[pl.ds(r0, 640), :])
                else:
                    pltpu.sync_copy(accs.at[pl.ds(r0, 640), :],
                                    agg.at[f].at[pl.ds(r0, 640), :])
                plsc.subcore_barrier()

    return k


def _pooling():
    @functools.partial(
        pl.kernel,
        out_type=jax.ShapeDtypeStruct((NG, 2 * HID), jnp.float32),
        mesh=_mesh(),
        compiler_params=pltpu.CompilerParams(needs_layout_passes=False),
        scratch_types=[pltpu.VMEM((272,), jnp.int32),
                       pltpu.VMEM((272,), jnp.int32),
                       pltpu.VMEM((8, 128), jnp.float32),
                       pltpu.VMEM((2 * HID,), jnp.float32),
                       pltpu.SemaphoreType.DMA])
    def k(hf, starts_h, cnts_h, out, sv, cv, buf, rowbuf, sem):
        wid = _sc_wid()
        pltpu.sync_copy(starts_h, sv)
        pltpu.sync_copy(cnts_h, cv)
        srow = sv[pl.ds(8 * wid, 16)]
        crow = cv[pl.ds(8 * wid, 16)]
        for j in range(8):
            st = srow[j]
            cn = crow[j]
            st8 = pl.multiple_of(st & (-8), 8)
            dlt = st - st8
            nb = (cn + dlt + 7) >> 3
            cnf = jnp.maximum(cn.astype(jnp.float32), 1.0)
            invv = 1.0 / jnp.full((16,), cnf, jnp.float32)
            nz = cn > 0
            for f in range(NF):
                def blk(b, carry):
                    pltpu.async_copy(
                        hf.at[f].at[pl.ds(st8 + 8 * b, 8), :], buf,
                        sem).wait()
                    out_c = []
                    for q in range(8):
                        mx, sm = carry[2 * q], carry[2 * q + 1]
                        for r in range(8):
                            idx = 8 * b + r
                            valid = (idx >= dlt) & (idx < dlt + cn)
                            vm = jnp.full((16,), valid, jnp.bool_)
                            v = buf[r, pl.ds(16 * q, 16)]
                            mx = jnp.maximum(mx, jnp.where(vm, v, -3e38))
                            sm = sm + jnp.where(vm, v, 0.0)
                        out_c += [mx, sm]
                    return tuple(out_c)

                init = []
                for q in range(8):
                    init += [jnp.full((16,), -3e38, jnp.float32),
                             jnp.zeros((16,), jnp.float32)]
                res = lax.fori_loop(0, nb, blk, tuple(init))
                nzv = jnp.full((16,), nz, jnp.bool_)
                for q in range(8):
                    gmax = jnp.where(nzv, res[2 * q], 0.0)
                    gmax = jnp.where(nzv, gmax, 0.0)
                    gmean = res[2 * q + 1] * invv
                    rowbuf[pl.ds(128 * f + 16 * q, 16)] = gmax
                    rowbuf[pl.ds(HID + 128 * f + 16 * q, 16)] = gmean
            pltpu.sync_copy(rowbuf, out.at[8 * wid + j])

    return k


def _gatv2_layer(xc, src, dst, wl, wr, attf, bias, heads, with_deg,
                 in_bias=None, in_act=None, x4=None):
    kc = xc.shape[0]
    wl_r = wl.reshape(kc, 128, NF, 128).transpose(0, 2, 1, 3)
    wr_r = wr.reshape(kc, 128, NF, 128).transpose(0, 2, 1, 3)
    xlc = _mm_chunked(xc, wl_r, bias=in_bias, act=in_act, x4=x4)
    xrc = _mm_chunked(xc, wr_r, bias=in_bias, act=in_act, x4=x4)
    attc = attf.reshape(NF, 128)
    u1 = _bounds(xlc, attc, heads)
    u2 = _bounds(xrc, attc, heads)
    mx = _colmax(u1)
    s0 = _s0_table(u2, mx)
    l_e, d0p = _edge_logits(heads, with_deg)(
        xlc, xrc, s0, attf, src, dst)
    tabs = _lsr_table(d0p, heads, with_deg)
    lsr = tabs[0]
    dinv = tabs[1] if with_deg else None
    alpha = _alpha_pass()(l_e, lsr, dst)
    agg, agg4b = _aggregate(heads)(xlc, alpha, src.reshape(-1, 96),
                                   dst.reshape(-1, 96))
    return agg, agg4b, dinv


def kernel(x, edge_index, batch, Wl1, Wr1, att1, b1, Wl2, Wr2, att2, b2,
           W3, b3):
    f32 = jnp.float32
    x = x.astype(f32)
    xp = jnp.pad(x, ((0, NP - N), (0, 0))).reshape(1, NP, 128)
    sl = jnp.arange(N, dtype=jnp.int32)
    src = jnp.concatenate([edge_index[0].astype(jnp.int32), sl,
                           jnp.full((EP - E0 - N,), N, jnp.int32)])
    dst = jnp.concatenate([edge_index[1].astype(jnp.int32), sl,
                           jnp.full((EP - E0 - N,), N, jnp.int32)])

    agg1, agg1b, dinv = _gatv2_layer(xp, src, dst, Wl1, Wr1,
                                     att1.reshape(-1), b1, 10, True)
    b1c = b1.reshape(NF, 128)
    agg2, agg2b, _ = _gatv2_layer(agg1, src, dst, Wl2, Wr2,
                                  att2.reshape(-1), b2, 1, False,
                                  in_bias=b1c, in_act="elu", x4=agg1b)
    b2c = b2.reshape(NF, 128)
    w3_r = W3.reshape(NF, 128, NF, 128).transpose(0, 2, 1, 3)
    hws = _mm_chunked(agg2, w3_r, bias=b2c, out_scale=dinv, x4=agg2b)
    dummy_a = jnp.zeros((128,), jnp.float32)
    aggg, agggb = _aggregate(0)(hws, dummy_a, src.reshape(-1, 96),
                                dst.reshape(-1, 96))
    hf = _final_feat(aggg, agggb, dinv, b3.reshape(NF, 128))

    batchp = jnp.pad(batch.astype(jnp.int32), (0, NP - N),
                     constant_values=NG)
    cnt, starts = _graph_tables(batchp.reshape(NP, 1))
    starts_p = jnp.pad(starts.reshape(NG), (0, 16))
    cnts_p = jnp.pad(cnt.reshape(NG), (0, 16))
    return _pooling()(hf, starts_p, cnts_p)


# speedup vs baseline: 7.1822x; 1.0332x over previous
import functools

import jax
import jax.numpy as jnp
from jax import lax
from jax.experimental import pallas as pl
from jax.experimental.pallas import tpu as pltpu
from jax.experimental.pallas import tpu_sc as plsc

N = 10000
NP = 10240
E0 = 320000
EP = 331776
HID = 640
NF = 5
NG = 256
BM = 512
def _mesh():
    return plsc.VectorSubcoreMesh(core_axis_name="c", subcore_axis_name="s")
_OWNER = (0, 0, 1, 1, 0)


def _mm_chunked(xc, wr, bias=None, act=None, out_scale=None, x4=None):
    kc = xc.shape[0]
    grid = (NF, NP // BM, kc)

    def body(*refs):
        refs = list(refs)
        x_ref = refs.pop(0)
        w_ref = refs.pop(0)
        e_ref = refs.pop(0) if x4 is not None else None
        b_ref = refs.pop(0) if bias is not None else None
        s_ref = refs.pop(0) if out_scale is not None else None
        o_ref = refs.pop(0)
        i = pl.program_id(1)
        ki = pl.program_id(2)
        x = x_ref[0]
        if e_ref is not None:
            x = x + jnp.where(ki == 4, 1.0, 0.0) * e_ref[...]
        if b_ref is not None:
            x = x + b_ref[0]
        if act == "elu":
            x = jnp.where(x > 0, x, jnp.exp(x) - 1.0)
        acc = jnp.dot(x, w_ref[0, 0],
                      preferred_element_type=jnp.float32)

        @pl.when(ki == 0)
        def _():
            o_ref[0] = jnp.zeros_like(o_ref[0])

        o_ref[0] += acc

        @pl.when(ki == kc - 1)
        def _():
            r = o_ref[0]
            if s_ref is not None:
                r = r * s_ref[...]
            rows = i * BM + lax.broadcasted_iota(jnp.int32, (BM, 128), 0)
            o_ref[0] = jnp.where(rows < N, r, 0.0)

    specs = [
        pl.BlockSpec((1, BM, 128), lambda j, i, ki: (ki, i, 0)),
        pl.BlockSpec((1, 1, 128, 128), lambda j, i, ki: (ki, j, 0, 0)),
    ]
    args = [xc, wr]
    if x4 is not None:
        specs.append(pl.BlockSpec((BM, 128), lambda j, i, ki: (i, 0)))
        args.append(x4)
    if bias is not None:
        specs.append(pl.BlockSpec((1, 1, 128), lambda j, i, ki: (ki, 0, 0)))
        args.append(bias.reshape(-1, 1, 128))
    if out_scale is not None:
        specs.append(pl.BlockSpec((BM, 128), lambda j, i, ki: (i, 0)))
        args.append(out_scale)
    return pl.pallas_call(
        body, grid=grid, in_specs=specs,
        out_specs=pl.BlockSpec((1, BM, 128), lambda j, i, ki: (j, i, 0)),
        out_shape=jax.ShapeDtypeStruct((NF, NP, 128), jnp.float32),
    )(*args)


def _bounds(xc, attc, heads):
    def body(x_ref, a_ref, o_ref):
        cols = []
        if heads == 1:
            tot = jnp.zeros((BM,), jnp.float32)
            for f in range(NF):
                t = x_ref[f] * a_ref[f][None, :]
                tot = tot + jnp.sum(jnp.maximum(t, 0.2 * t), axis=-1)
            cols.append(tot)
        else:
            for h in range(heads):
                f, sl = (64 * h) // 128, (64 * h) % 128
                t = x_ref[f, :, sl:sl + 64] * a_ref[f, sl:sl + 64][None, :]
                cols.append(jnp.sum(jnp.maximum(t, 0.2 * t), axis=-1))
        stacked = jnp.stack(cols, axis=-1)
        o_ref[...] = jnp.concatenate(
            [stacked, jnp.zeros((BM, 128 - len(cols)), jnp.float32)],
            axis=-1)

    return pl.pallas_call(
        body, grid=(NP // BM,),
        in_specs=[pl.BlockSpec((NF, BM, 128), lambda i: (0, i, 0)),
                  pl.BlockSpec((NF, 128), lambda i: (0, 0))],
        out_specs=pl.BlockSpec((BM, 128), lambda i: (i, 0)),
        out_shape=jax.ShapeDtypeStruct((NP, 128), jnp.float32),
    )(xc, attc)


def _colmax(u):
    def body(u_ref, o_ref):
        i = pl.program_id(0)

        @pl.when(i == 0)
        def _():
            o_ref[...] = jnp.full((1, 128), -3e38, jnp.float32)

        o_ref[...] = jnp.maximum(o_ref[...],
                                 jnp.max(u_ref[...], axis=0, keepdims=True))

    return pl.pallas_call(
        body, grid=(NP // BM,),
        in_specs=[pl.BlockSpec((BM, 128), lambda i: (i, 0))],
        out_specs=pl.BlockSpec((1, 128), lambda i: (0, 0)),
        out_shape=jax.ShapeDtypeStruct((1, 128), jnp.float32),
    )(u)


def _s0_table(u2, mx):
    def body(u_ref, m_ref, o_ref):
        o_ref[...] = u_ref[...] + m_ref[...]

    return pl.pallas_call(
        body, grid=(NP // BM,),
        in_specs=[pl.BlockSpec((BM, 128), lambda i: (i, 0)),
                  pl.BlockSpec((1, 128), lambda i: (0, 0))],
        out_specs=pl.BlockSpec((BM, 128), lambda i: (i, 0)),
        out_shape=jax.ShapeDtypeStruct((NP, 128), jnp.float32),
    )(u2, mx)


def _lsr_table(d0p, heads, with_dinv):
    n_out = 2 if with_dinv else 1

    def body(d_ref, o_ref, *maybe_dinv):
        d0 = d_ref[0] + d_ref[1]
        ls = jnp.log(jnp.maximum(d0, 1e-35))
        den = d0 / jnp.maximum(d0, 1e-35)
        r = 1.0 / (den + 1e-16)
        colv = lax.broadcasted_iota(jnp.int32, (BM, 128), 1)
        r = jnp.where(colv < heads, r, 0.0)
        o_ref[...] = jnp.concatenate(
            [ls[:, :16], r[:, :16], jnp.zeros((BM, 96), jnp.float32)],
            axis=1)
        if with_dinv:
            dv = maybe_dinv[0]
            deg = d0[:, 10]
            dinv = 1.0 / jnp.sqrt(jnp.maximum(deg, 1.0))
            dv[...] = jnp.broadcast_to(dinv[:, None], (BM, 128))

    out_shape = [jax.ShapeDtypeStruct((NP, 128), jnp.float32)] * n_out
    res = pl.pallas_call(
        body, grid=(NP // BM,),
        in_specs=[pl.BlockSpec((2, BM, 128), lambda i: (0, i, 0))],
        out_specs=[pl.BlockSpec((BM, 128), lambda i: (i, 0))] * n_out,
        out_shape=out_shape,
    )(d0p)
    return res if with_dinv else (res[0],)


def _final_feat(aggc, agg4b, dinv, b3c):
    def body(a_ref, e_ref, d_ref, b_ref, o_ref):
        f = pl.program_id(0)
        i = pl.program_id(1)
        a = a_ref[0] + jnp.where(f == 4, 1.0, 0.0) * e_ref[...]
        r = jnp.maximum(a * d_ref[...] + b_ref[0], 0.0)
        rows = i * BM + lax.broadcasted_iota(jnp.int32, (BM, 128), 0)
        o_ref[0] = jnp.where(rows < N, r, 0.0)

    return pl.pallas_call(
        body, grid=(NF, NP // BM),
        in_specs=[pl.BlockSpec((1, BM, 128), lambda f, i: (f, i, 0)),
                  pl.BlockSpec((BM, 128), lambda f, i: (i, 0)),
                  pl.BlockSpec((BM, 128), lambda f, i: (i, 0)),
                  pl.BlockSpec((1, 1, 128), lambda f, i: (f, 0, 0))],
        out_specs=pl.BlockSpec((1, BM, 128), lambda f, i: (f, i, 0)),
        out_shape=jax.ShapeDtypeStruct((NF, NP, 128), jnp.float32),
    )(aggc, agg4b, dinv, b3c.reshape(NF, 1, 128))


def _graph_tables(batch2d):
    def body(b_ref, c_ref, s_ref):
        i = pl.program_id(0)

        @pl.when(i == 0)
        def _():
            c_ref[...] = jnp.zeros((1, NG), jnp.int32)
            s_ref[...] = jnp.zeros((1, NG), jnp.int32)

        b = b_ref[...]
        g = lax.broadcasted_iota(jnp.int32, (BM, NG), 1)
        c_ref[...] += jnp.sum((b == g).astype(jnp.int32), axis=0,
                              keepdims=True)
        s_ref[...] += jnp.sum((b < g).astype(jnp.int32), axis=0,
                              keepdims=True)

    return pl.pallas_call(
        body, grid=(NP // BM,),
        in_specs=[pl.BlockSpec((BM, 1), lambda i: (i, 0))],
        out_specs=[pl.BlockSpec((1, NG), lambda i: (0, 0))] * 2,
        out_shape=[jax.ShapeDtypeStruct((1, NG), jnp.int32)] * 2,
    )(batch2d)


def _sc_wid():
    return lax.axis_index("s") * 2 + lax.axis_index("c")


def _edge_logits(heads, with_deg):
    CA = 24
    PT = EP // 32
    nch = PT // CA
    ch_head = HID // heads

    scratch = [pltpu.VMEM((HID,), jnp.float32)]
    scratch += [pltpu.VMEM((CA, 128), jnp.float32) for _ in range(10)]
    scratch += [pltpu.VMEM((CA, 128), jnp.float32)]
    scratch += [pltpu.VMEM((CA,), jnp.int32),
                pltpu.VMEM((CA,), jnp.int32),
                pltpu.VMEM((CA,), jnp.int32),
                pltpu.VMEM((CA,), jnp.int32),
                pltpu.VMEM((CA * 16,), jnp.float32),
                pltpu.VMEM((CA, 128), jnp.float32),
                pltpu.VMEM((16, 128), jnp.float32),
                pltpu.VMEM_SHARED((NP, 128), jnp.float32),
                pltpu.SemaphoreType.DMA]

    @functools.partial(
        pl.kernel,
        out_type=[jax.ShapeDtypeStruct((EP * 16,), jnp.float32),
                  jax.ShapeDtypeStruct((2, NP, 128), jnp.float32)],
        mesh=_mesh(),
        compiler_params=pltpu.CompilerParams(needs_layout_passes=False),
        scratch_types=scratch)
    def k(xlc, xrc, s0t, att_h, srca, dsta, l_out, d0_out,
          attv, xb0, xb1, xb2, xb3, xb4, rb0, rb1, rb2, rb3, rb4, s0b,
          idxs, idxd, idxs2, idxd2, lbuf, pbuf, zbuf, d0s, sem):
        cc = lax.axis_index("c")
        wid = _sc_wid()
        xlb = [xb0, xb1, xb2, xb3, xb4]
        xrb = [rb0, rb1, rb2, rb3, rb4]
        lane = lax.iota(jnp.int32, 16)

        pltpu.sync_copy(att_h, attv)
        for i in range(16):
            for q in range(8):
                zbuf[i, pl.ds(16 * q, 16)] = jnp.zeros((16,), jnp.float32)
        for b in range(40):
            r0 = lax.axis_index("s") * 640 + b * 16
            pltpu.sync_copy(zbuf, d0s.at[pl.ds(r0, 16), :])
        for i in range(CA):
            for q in range(1, 8):
                pbuf[i, pl.ds(16 * q, 16)] = jnp.zeros((16,), jnp.float32)
        plsc.subcore_barrier()

        tile_base = wid * PT

        def _ibase(ci):
            return pl.multiple_of(tile_base + ci * CA, 8)

        pltpu.make_async_copy(srca.at[pl.ds(_ibase(0), CA)], idxs,
                              sem).start()
        pltpu.make_async_copy(dsta.at[pl.ds(_ibase(0), CA)], idxd,
                              sem).start()

        def half_chunk(ci, ib_s, ib_d, nb_s, nb_d):
            base = _ibase(ci)
            pltpu.make_async_copy(srca.at[pl.ds(base, CA)], ib_s,
                                  sem).wait()
            pltpu.make_async_copy(dsta.at[pl.ds(base, CA)], ib_d,
                                  sem).wait()
            descs = []
            for f in range(NF):
                descs.append(pltpu.async_copy(xlc.at[f].at[ib_s],
                                              xlb[f], sem))
                descs.append(pltpu.async_copy(xrc.at[f].at[ib_d],
                                              xrb[f], sem))
            descs.append(pltpu.async_copy(s0t.at[ib_d], s0b, sem))

            @pl.when(ci + 1 < nch)
            def _():
                nbase = _ibase(ci + 1)
                pltpu.make_async_copy(srca.at[pl.ds(nbase, CA)], nb_s,
                                      sem).start()
                pltpu.make_async_copy(dsta.at[pl.ds(nbase, CA)], nb_d,
                                      sem).start()
            for d in descs:
                d.wait()

            def edge(i, _):
                tots = []
                for h in range(heads):
                    acc = None
                    for q in range(ch_head // 16):
                        g = h * ch_head + q * 16
                        f, off = g // 128, g % 128
                        a = xlb[f][i, pl.ds(off, 16)]
                        b = xrb[f][i, pl.ds(off, 16)]
                        z = a + b
                        w = jnp.maximum(z, 0.2 * z) * attv[pl.ds(g, 16)]
                        acc = w if acc is None else acc + w
                    tots.append(jnp.sum(acc))
                lrow = jnp.zeros((16,), jnp.float32)
                for h in range(heads):
                    lrow = jnp.where(lane == h,
                                     jnp.full((16,), tots[h], jnp.float32),
                                     lrow)
                s0r = s0b[i, pl.ds(0, 16)]
                hm = lane < heads
                ls = jnp.where(hm, lrow - s0r, 0.0)
                p = jnp.where(hm, jnp.exp(ls), 0.0)
                if with_deg:
                    p = p + jnp.where(lane == 10, 1.0, 0.0)
                lbuf[pl.ds(i * 16, 16)] = ls
                pbuf[i, pl.ds(0, 16)] = p
                return 0

            lax.fori_loop(0, CA, edge, 0)
            pltpu.sync_copy(lbuf, l_out.at[pl.ds(base * 16, CA * 16)])
            pltpu.sync_copy(pbuf, d0s.at[ib_d], add=True)

        def pair(g, _):
            half_chunk(2 * g, idxs, idxd, idxs2, idxd2)
            half_chunk(2 * g + 1, idxs2, idxd2, idxs, idxd)
            return 0

        lax.fori_loop(0, nch // 2, pair, 0)
        plsc.subcore_barrier()
        r0 = lax.axis_index("s") * 640
        pltpu.sync_copy(d0s.at[pl.ds(r0, 640), :],
                        d0_out.at[cc].at[pl.ds(r0, 640), :])

    return k


def _alpha_pass():
    CB = 64
    PT = EP // 32
    nch = PT // CB

    @functools.partial(
        pl.kernel,
        out_type=jax.ShapeDtypeStruct((EP * 16,), jnp.float32),
        mesh=_mesh(),
        compiler_params=pltpu.CompilerParams(needs_layout_passes=False),
        scratch_types=[pltpu.VMEM((CB * 16,), jnp.float32),
                       pltpu.VMEM((CB, 128), jnp.float32),
                       pltpu.VMEM((CB,), jnp.int32),
                       pltpu.VMEM((CB * 16,), jnp.float32),
                       pltpu.SemaphoreType.DMA])
    def k(l_in, lsr, dsta, a_out, lb, lsrb, idxd, ab, sem):
        wid = _sc_wid()
        tile_base = wid * PT

        def chunk(ci, _):
            base = pl.multiple_of(tile_base + ci * CB, 8)
            pltpu.sync_copy(dsta.at[pl.ds(base, CB)], idxd)
            d1 = pltpu.async_copy(l_in.at[pl.ds(base * 16, CB * 16)], lb,
                                  sem)
            d2 = pltpu.async_copy(lsr.at[idxd], lsrb, sem)
            d1.wait()
            d2.wait()

            def edge(i, _):
                ls = lsrb[i, pl.ds(0, 16)]
                r = lsrb[i, pl.ds(16, 16)]
                ab[pl.ds(i * 16, 16)] = (
                    jnp.exp(lb[pl.ds(i * 16, 16)] - ls) * r)
                return 0

            lax.fori_loop(0, CB, edge, 0)
            pltpu.sync_copy(ab, a_out.at[pl.ds(base * 16, CB * 16)])
            return 0

        lax.fori_loop(0, nch, chunk, 0)

    return k


def _aggregate(heads):
    CC = 96
    SB = 8
    PT = EP // 16
    ntch = PT // CC
    nsb = ntch // SB
    has_alpha = heads > 0

    scratch = [pltpu.VMEM((CC, 128), jnp.float32),
               pltpu.VMEM((CC, 128), jnp.float32),
               pltpu.VMEM((SB * CC * 16,), jnp.float32),
               pltpu.VMEM((SB, CC), jnp.int32),
               pltpu.VMEM((SB, CC), jnp.int32),
               pltpu.VMEM((16, 128), jnp.float32),
               pltpu.VMEM_SHARED((NP, 128), jnp.float32),
               pltpu.SemaphoreType.DMA,
               pltpu.SemaphoreType.DMA]

    @functools.partial(
        pl.kernel,
        out_type=[jax.ShapeDtypeStruct((NF, NP, 128), jnp.float32),
                  jax.ShapeDtypeStruct((NP, 128), jnp.float32)],
        mesh=_mesh(),
        compiler_params=pltpu.CompilerParams(needs_layout_passes=False),
        scratch_types=scratch)
    def k(xlc, alpha, src_r, dst_r, agg, agg4b,
          xb0, xb1, alb, isb, idb, zbuf, accs, semg, sems):
        cc = lax.axis_index("c")
        sid = lax.axis_index("s")
        xbs = [xb0, xb1]

        for i in range(16):
            for q in range(8):
                zbuf[i, pl.ds(16 * q, 16)] = jnp.zeros((16,), jnp.float32)

        for f in range(NF):
            @pl.when((cc == _OWNER[f]) | (f == 4))
            def _(f=f):
                for b in range(40):
                    r0 = sid * 640 + b * 16
                    pltpu.sync_copy(zbuf, accs.at[pl.ds(r0, 16), :])
                plsc.subcore_barrier()

                def sblock(sb, _):
                    row0 = pl.multiple_of(sid * ntch + sb * SB, 8)
                    pltpu.sync_copy(src_r.at[pl.ds(row0, SB), :], isb)
                    pltpu.sync_copy(dst_r.at[pl.ds(row0, SB), :], idb)
                    if has_alpha:
                        eb = pl.multiple_of(row0 * CC * 16, 8)
                        pltpu.sync_copy(alpha.at[pl.ds(eb, SB * CC * 16)],
                                        alb)

                    def g_desc(kk, buf):
                        return pltpu.make_async_copy(
                            xlc.at[f].at[isb.at[kk]], buf, semg)

                    def s_desc(kk, buf):
                        return pltpu.make_async_copy(
                            buf, accs.at[idb.at[kk]], sems)

                    g_desc(0, xbs[0]).start()
                    last_sc = [None, None]
                    for kk in range(SB):
                        p = kk % 2
                        if kk + 1 < SB:
                            q = (kk + 1) % 2
                            if last_sc[q] is not None:
                                s_desc(last_sc[q], xbs[q]).wait()
                            g_desc(kk + 1, xbs[q]).start()
                        g_desc(kk, xbs[p]).wait()
                        if has_alpha:
                            xb = xbs[p]

                            def edge(i, _, kk=kk, xb=xb):
                                arow = alb[pl.ds((kk * CC + i) * 16, 16)]
                                if heads == 1:
                                    a0 = arow[0]
                                    a1 = a0
                                else:
                                    a0 = arow[2 * f]
                                    a1 = arow[2 * f + 1]
                                av0 = jnp.full((16,), a0, jnp.float32)
                                av1 = jnp.full((16,), a1, jnp.float32)
                                for q2 in range(8):
                                    av = av0 if q2 < 4 else av1
                                    xb[i, pl.ds(16 * q2, 16)] = (
                                        xb[i, pl.ds(16 * q2, 16)] * av)
                                return 0

                            lax.fori_loop(0, CC, edge, 0)
                        pltpu.async_copy(xbs[p], accs.at[idb.at[kk]],
                                         sems, add=True)
                        last_sc[p] = kk
                    for q in (0, 1):
                        if last_sc[q] is not None:
                            s_desc(last_sc[q], xbs[q]).wait()
                    return 0

                if f == 4:
                    lo = jnp.where(cc == 0, 0, 13)
                    hi = jnp.where(cc == 0, 13, nsb)
                    lax.fori_loop(lo, hi, sblock, 0)
                else:
                    lax.fori_loop(0, nsb, sblock, 0)
                plsc.subcore_barrier()
                r0 = sid * 640
                if f == 4:
                    @pl.when(cc == 0)
                    def _():
                        pltpu.sync_copy(accs.at[pl.ds(r0, 640), :],
                                        agg.at[f].at[pl.ds(r0, 640), :])

                    @pl.when(cc == 1)
                    def _():
                        pltpu.sync_copy(accs.at[pl.ds(r0, 640), :],
                                        agg4b.at[pl.ds(r0, 640), :])
                else:
                    pltpu.sync_copy(accs.at[pl.ds(r0, 640), :],
                                    agg.at[f].at[pl.ds(r0, 640), :])
                plsc.subcore_barrier()

    return k


def _pooling():
    @functools.partial(
        pl.kernel,
        out_type=jax.ShapeDtypeStruct((NG, 2 * HID), jnp.float32),
        mesh=_mesh(),
        compiler_params=pltpu.CompilerParams(needs_layout_passes=False),
        scratch_types=[pltpu.VMEM((272,), jnp.int32),
                       pltpu.VMEM((272,), jnp.int32),
                       pltpu.VMEM((8, 128), jnp.float32),
                       pltpu.VMEM((2 * HID,), jnp.float32),
                       pltpu.SemaphoreType.DMA])
    def k(hf, starts_h, cnts_h, out, sv, cv, buf, rowbuf, sem):
        wid = _sc_wid()
        pltpu.sync_copy(starts_h, sv)
        pltpu.sync_copy(cnts_h, cv)
        srow = sv[pl.ds(8 * wid, 16)]
        crow = cv[pl.ds(8 * wid, 16)]
        for j in range(8):
            st = srow[j]
            cn = crow[j]
            st8 = pl.multiple_of(st & (-8), 8)
            dlt = st - st8
            nb = (cn + dlt + 7) >> 3
            cnf = jnp.maximum(cn.astype(jnp.float32), 1.0)
            invv = 1.0 / jnp.full((16,), cnf, jnp.float32)
            nz = cn > 0
            for f in range(NF):
                def blk(b, carry):
                    pltpu.async_copy(
                        hf.at[f].at[pl.ds(st8 + 8 * b, 8), :], buf,
                        sem).wait()
                    out_c = []
                    for q in range(8):
                        mx, sm = carry[2 * q], carry[2 * q + 1]
                        for r in range(8):
                            idx = 8 * b + r
                            valid = (idx >= dlt) & (idx < dlt + cn)
                            vm = jnp.full((16,), valid, jnp.bool_)
                            v = buf[r, pl.ds(16 * q, 16)]
                            mx = jnp.maximum(mx, jnp.where(vm, v, -3e38))
                            sm = sm + jnp.where(vm, v, 0.0)
                        out_c += [mx, sm]
                    return tuple(out_c)

                init = []
                for q in range(8):
                    init += [jnp.full((16,), -3e38, jnp.float32),
                             jnp.zeros((16,), jnp.float32)]
                res = lax.fori_loop(0, nb, blk, tuple(init))
                nzv = jnp.full((16,), nz, jnp.bool_)
                for q in range(8):
                    gmax = jnp.where(nzv, res[2 * q], 0.0)
                    gmax = jnp.where(nzv, gmax, 0.0)
                    gmean = res[2 * q + 1] * invv
                    rowbuf[pl.ds(128 * f + 16 * q, 16)] = gmax
                    rowbuf[pl.ds(HID + 128 * f + 16 * q, 16)] = gmean
            pltpu.sync_copy(rowbuf, out.at[8 * wid + j])

    return k


def _gatv2_layer(xc, src, dst, wl, wr, attf, bias, heads, with_deg,
                 in_bias=None, in_act=None, x4=None):
    kc = xc.shape[0]
    wl_r = wl.reshape(kc, 128, NF, 128).transpose(0, 2, 1, 3)
    wr_r = wr.reshape(kc, 128, NF, 128).transpose(0, 2, 1, 3)
    xlc = _mm_chunked(xc, wl_r, bias=in_bias, act=in_act, x4=x4)
    xrc = _mm_chunked(xc, wr_r, bias=in_bias, act=in_act, x4=x4)
    attc = attf.reshape(NF, 128)
    u1 = _bounds(xlc, attc, heads)
    u2 = _bounds(xrc, attc, heads)
    mx = _colmax(u1)
    s0 = _s0_table(u2, mx)
    l_e, d0p = _edge_logits(heads, with_deg)(
        xlc, xrc, s0, attf, src, dst)
    tabs = _lsr_table(d0p, heads, with_deg)
    lsr = tabs[0]
    dinv = tabs[1] if with_deg else None
    alpha = _alpha_pass()(l_e, lsr, dst)
    agg, agg4b = _aggregate(heads)(xlc, alpha, src.reshape(-1, 96),
                                   dst.reshape(-1, 96))
    return agg, agg4b, dinv


def kernel(x, edge_index, batch, Wl1, Wr1, att1, b1, Wl2, Wr2, att2, b2,
           W3, b3):
    f32 = jnp.float32
    x = x.astype(f32)
    xp = jnp.pad(x, ((0, NP - N), (0, 0))).reshape(1, NP, 128)
    sl = jnp.arange(N, dtype=jnp.int32)
    src = jnp.concatenate([edge_index[0].astype(jnp.int32), sl,
                           jnp.full((EP - E0 - N,), N, jnp.int32)])
    dst = jnp.concatenate([edge_index[1].astype(jnp.int32), sl,
                           jnp.full((EP - E0 - N,), N, jnp.int32)])

    agg1, agg1b, dinv = _gatv2_layer(xp, src, dst, Wl1, Wr1,
                                     att1.reshape(-1), b1, 10, True)
    b1c = b1.reshape(NF, 128)
    agg2, agg2b, _ = _gatv2_layer(agg1, src, dst, Wl2, Wr2,
                                  att2.reshape(-1), b2, 1, False,
                                  in_bias=b1c, in_act="elu", x4=agg1b)
    b2c = b2.reshape(NF, 128)
    w3_r = W3.reshape(NF, 128, NF, 128).transpose(0, 2, 1, 3)
    hws = _mm_chunked(agg2, w3_r, bias=b2c, out_scale=dinv, x4=agg2b)
    dummy_a = jnp.zeros((128,), jnp.float32)
    aggg, agggb = _aggregate(0)(hws, dummy_a, src.reshape(-1, 96),
                                dst.reshape(-1, 96))
    hf = _final_feat(aggg, agggb, dinv, b3.reshape(NF, 128))

    batchp = jnp.pad(batch.astype(jnp.int32), (0, NP - N),
                     constant_values=NG)
    cnt, starts = _graph_tables(batchp.reshape(NP, 1))
    starts_p = jnp.pad(starts.reshape(NG), (0, 16))
    cnts_p = jnp.pad(cnt.reshape(NG), (0, 16))
    return _pooling()(hf, starts_p, cnts_p)
